# baseline (speedup 1.0000x reference)
import functools

import jax
import jax.numpy as jnp
from jax import lax
from jax.experimental import pallas as pl
from jax.experimental.pallas import tpu as pltpu
from jax.experimental.pallas import tpu_sc as plsc

N_NODES = 10000
N_EDGES = 320000
H = 128
DEPTH = 3

NC = 2
NS = 16
NW = NC * NS
EPW = N_EDGES // NW
CH = 80
NCHUNK = EPW // CH
N_PAD = 10112
RPS = N_PAD // NS

EBLK = 2000
NBLK = 2000

_mesh = plsc.VectorSubcoreMesh(core_axis_name="c", subcore_axis_name="s")


K = 5


def _make_gather(n_edges, ch):
    epw = n_edges // NW
    n_outer = epw // ch // K

    @functools.partial(
        pl.kernel,
        out_type=jax.ShapeDtypeStruct((n_edges, H), jnp.float32),
        mesh=_mesh,
        scratch_types=[
            pltpu.VMEM((epw,), jnp.int32),
            pltpu.VMEM((epw,), jnp.int32),
            pltpu.VMEM((K, ch, H), jnp.float32),
            pltpu.SemaphoreType.DMA,
            pltpu.SemaphoreType.DMA,
            pltpu.SemaphoreType.DMA,
        ],
    )
    def gather(xni_hbm, xno_hbm, row_hbm, col_hbm, out_hbm,
               ridx_v, cidx_v, bufs, sem_g, sem_a, sem_w):
        wid = lax.axis_index("s") * NC + lax.axis_index("c")
        base = wid * epw
        pltpu.sync_copy(row_hbm.at[pl.ds(base, epw)], ridx_v)
        pltpu.sync_copy(col_hbm.at[pl.ds(base, epw)], cidx_v)

        def outer(g, _):
            @pl.when(g > 0)
            def _():
                for b in range(K):
                    off = base + ((g - 1) * K + b) * ch
                    pltpu.make_async_copy(
                        bufs.at[b], out_hbm.at[pl.ds(off, ch)], sem_w).wait()
            ds = []
            for b in range(K):
                c = (g * K + b) * ch
                ds.append(pltpu.async_copy(
                    xni_hbm.at[ridx_v.at[pl.ds(c, ch)]], bufs.at[b], sem_g))
            for d in ds:
                d.wait()
            ds = []
            for b in range(K):
                c = (g * K + b) * ch
                ds.append(pltpu.async_copy(
                    xno_hbm.at[cidx_v.at[pl.ds(c, ch)]], bufs.at[b], sem_a,
                    add=True))
            for d in ds:
                d.wait()
            for b in range(K):
                off = base + (g * K + b) * ch
                pltpu.async_copy(bufs.at[b], out_hbm.at[pl.ds(off, ch)],
                                 sem_w)
            return 0

        lax.fori_loop(0, n_outer, outer, 0)
        for b in range(K):
            off = base + ((n_outer - 1) * K + b) * ch
            pltpu.make_async_copy(
                bufs.at[b], out_hbm.at[pl.ds(off, ch)], sem_w).wait()

    return gather


def _make_scatter(n_edges, ch):
    epw = n_edges // NW
    n_outer = epw // ch // K

    @functools.partial(
        pl.kernel,
        out_type=jax.ShapeDtypeStruct((NC, N_PAD, H), jnp.float32),
        mesh=_mesh,
        scratch_types=[
            pltpu.VMEM((K, ch), jnp.int32),
            pltpu.VMEM((K, ch, H), jnp.float32),
            pltpu.VMEM_SHARED((N_PAD, H), jnp.float32),
            pltpu.SemaphoreType.DMA,
            pltpu.SemaphoreType.DMA,
            pltpu.SemaphoreType.DMA,
        ],
    )
    def scatter(m_hbm, col_hbm, zero_hbm, out_hbm, cidx2, bufs, acc_sh,
                sem_i, sem_m, sem_s):
        cid = lax.axis_index("c")
        sid = lax.axis_index("s")
        wid = sid * NC + cid
        pltpu.sync_copy(zero_hbm.at[pl.ds(sid * RPS, RPS)],
                        acc_sh.at[pl.ds(sid * RPS, RPS)])
        plsc.subcore_barrier()
        base = wid * epw

        def outer(g, _):
            @pl.when(g > 0)
            def _():
                for b in range(K):
                    pltpu.make_async_copy(
                        bufs.at[b], acc_sh.at[cidx2.at[b]], sem_s).wait()
            ds = []
            for b in range(K):
                off = base + (g * K + b) * ch
                ds.append(pltpu.async_copy(
                    col_hbm.at[pl.ds(off, ch)], cidx2.at[b], sem_i))
                ds.append(pltpu.async_copy(
                    m_hbm.at[pl.ds(off, ch)], bufs.at[b], sem_m))
            for d in ds:
                d.wait()
            for b in range(K):
                pltpu.async_copy(bufs.at[b], acc_sh.at[cidx2.at[b]], sem_s,
                                 add=True)
            return 0

        lax.fori_loop(0, n_outer, outer, 0)
        for b in range(K):
            pltpu.make_async_copy(bufs.at[b], acc_sh.at[cidx2.at[b]],
                                  sem_s).wait()
        plsc.subcore_barrier()
        pltpu.sync_copy(acc_sh.at[pl.ds(sid * RPS, RPS)],
                        out_hbm.at[cid, pl.ds(sid * RPS, RPS)])

    return scatter


E_HALF = N_EDGES // 2
_sc_gather_half = _make_gather(E_HALF, 40)
_sc_scatter_half = _make_scatter(E_HALF, 40)


def _dot(a, b):
    return jnp.dot(a, b, preferred_element_type=jnp.float32)


def _edge_body(scale_ref, edge_ref, fsum_ref, We_ref, eW1_ref, eW2_ref,
               n1W1_ref, n1W2_ref, be_ref, eb1_ref, eb2_ref, n1b1_ref,
               n1b2_ref, newe_ref, m_ref):
    e = edge_ref[...]
    h = jnp.maximum(_dot(e, We_ref[...]) + be_ref[...] + fsum_ref[...], 0.0)
    t = jnp.maximum(_dot(h, eW1_ref[...]) + eb1_ref[...], 0.0)
    ne = scale_ref[0] * e + _dot(t, eW2_ref[...]) + eb2_ref[...]
    u = jnp.maximum(_dot(ne, n1W1_ref[...]) + n1b1_ref[...], 0.0)
    newe_ref[...] = ne
    m_ref[...] = _dot(u, n1W2_ref[...]) + n1b2_ref[...]


_W_SPEC = pl.BlockSpec((H, H), lambda i: (0, 0))
_B_SPEC = pl.BlockSpec((1, H), lambda i: (0, 0))
_S_SPEC = pl.BlockSpec(memory_space=pltpu.SMEM)


def _make_edge_call(half):
    nblk = E_HALF // EBLK
    off = half * nblk

    return pl.pallas_call(
        _edge_body,
        grid=(nblk,),
        in_specs=[
            _S_SPEC,
            pl.BlockSpec((EBLK, H), lambda i: (i + off, 0)),
            pl.BlockSpec((EBLK, H), lambda i: (i, 0)),
            _W_SPEC, _W_SPEC, _W_SPEC, _W_SPEC, _W_SPEC,
            _B_SPEC, _B_SPEC, _B_SPEC, _B_SPEC, _B_SPEC,
        ],
        out_specs=[
            pl.BlockSpec((EBLK, H), lambda i: (i + off, 0)),
            pl.BlockSpec((EBLK, H), lambda i: (i, 0)),
        ],
        out_shape=[
            jax.ShapeDtypeStruct((N_EDGES, H), jnp.float32),
            jax.ShapeDtypeStruct((E_HALF, H), jnp.float32),
        ],
        input_output_aliases={1: 0},
    )


_edge_call_a = _make_edge_call(0)
_edge_call_b = _make_edge_call(1)


def _node_body_proj(scale_ref, agga_ref, aggb_ref, x_ref, n2W1_ref, n2W2_ref,
                    Wni_ref, Wno_ref, n2b1_ref, n2b2_ref, xnew_ref, xni_ref,
                    xno_ref):
    agg = agga_ref[0] + agga_ref[1] + aggb_ref[0] + aggb_ref[1]
    t = jnp.maximum(_dot(agg, n2W1_ref[...]) + n2b1_ref[...], 0.0)
    xn = scale_ref[0] * x_ref[...] + _dot(t, n2W2_ref[...]) + n2b2_ref[...]
    xnew_ref[...] = xn
    xni_ref[...] = _dot(xn, Wni_ref[...])
    xno_ref[...] = _dot(xn, Wno_ref[...])


def _node_body_last(scale_ref, agga_ref, aggb_ref, x_ref, n2W1_ref, n2W2_ref,
                    n2b1_ref, n2b2_ref, xnew_ref):
    agg = agga_ref[0] + agga_ref[1] + aggb_ref[0] + aggb_ref[1]
    t = jnp.maximum(_dot(agg, n2W1_ref[...]) + n2b1_ref[...], 0.0)
    xnew_ref[...] = scale_ref[0] * x_ref[...] + _dot(t, n2W2_ref[...]) \
        + n2b2_ref[...]


_AGG_SPEC = pl.BlockSpec((NC, NBLK, H), lambda i: (0, i, 0))
_N_SPEC = pl.BlockSpec((NBLK, H), lambda i: (i, 0))
_NODE_SHAPE = jax.ShapeDtypeStruct((N_NODES, H), jnp.float32)

_node_call_proj = pl.pallas_call(
    _node_body_proj,
    grid=(N_NODES // NBLK,),
    in_specs=[_S_SPEC, _AGG_SPEC, _AGG_SPEC, _N_SPEC,
              _W_SPEC, _W_SPEC, _W_SPEC, _W_SPEC, _B_SPEC, _B_SPEC],
    out_specs=[_N_SPEC, _N_SPEC, _N_SPEC],
    out_shape=[_NODE_SHAPE, _NODE_SHAPE, _NODE_SHAPE],
)

_node_call_last = pl.pallas_call(
    _node_body_last,
    grid=(N_NODES // NBLK,),
    in_specs=[_S_SPEC, _AGG_SPEC, _AGG_SPEC, _N_SPEC, _W_SPEC, _W_SPEC,
              _B_SPEC, _B_SPEC],
    out_specs=_N_SPEC,
    out_shape=_NODE_SHAPE,
)


def _proj_body(x_ref, Wni_ref, Wno_ref, xni_ref, xno_ref):
    x = x_ref[...]
    xni_ref[...] = _dot(x, Wni_ref[...])
    xno_ref[...] = _dot(x, Wno_ref[...])


_proj_call = pl.pallas_call(
    _proj_body,
    grid=(N_NODES // NBLK,),
    in_specs=[_N_SPEC, _W_SPEC, _W_SPEC],
    out_specs=[_N_SPEC, _N_SPEC],
    out_shape=[_NODE_SHAPE, _NODE_SHAPE],
)


def kernel(x, edge_index, edge_attr, We, Wni, Wno, eW1, eW2, n1W1, n1W2,
           n2W1, n2W2, be, eb1, eb2, n1b1, n1b2, n2b1, n2b2, eps_e, eps_n):
    row_a, col_a = edge_index[0, :E_HALF], edge_index[1, :E_HALF]
    row_b, col_b = edge_index[0, E_HALF:], edge_index[1, E_HALF:]
    zeros_nh = jnp.zeros((N_PAD, H), jnp.float32)

    xni, xno = _proj_call(x, Wni[0], Wno[0])
    for i in range(DEPTH):
        scale_e = (1.0 + eps_e[i]).reshape((1,))
        wargs = (We[i], eW1[i], eW2[i], n1W1[i], n1W2[i],
                 be[i].reshape(1, H), eb1[i].reshape(1, H),
                 eb2[i].reshape(1, H), n1b1[i].reshape(1, H),
                 n1b2[i].reshape(1, H))
        fs_a = _sc_gather_half(xni, xno, row_a, col_a)
        fs_b = _sc_gather_half(xni, xno, row_b, col_b)
        edge_attr, m_a = _edge_call_a(scale_e, edge_attr, fs_a, *wargs)
        agg_a = _sc_scatter_half(m_a, col_a, zeros_nh)
        edge_attr, m_b = _edge_call_b(scale_e, edge_attr, fs_b, *wargs)
        agg_b = _sc_scatter_half(m_b, col_b, zeros_nh)
        scale_n = (1.0 + eps_n[i]).reshape((1,))
        if i + 1 < DEPTH:
            x, xni, xno = _node_call_proj(
                scale_n, agg_a, agg_b, x, n2W1[i], n2W2[i], Wni[i + 1],
                Wno[i + 1], n2b1[i].reshape(1, H), n2b2[i].reshape(1, H))
        else:
            x = _node_call_last(
                scale_n, agg_a, agg_b, x, n2W1[i], n2W2[i],
                n2b1[i].reshape(1, H), n2b2[i].reshape(1, H))
    return (x, edge_attr)


# speedup vs baseline: 1.1255x; 1.1255x over previous
import functools

import jax
import jax.numpy as jnp
from jax import lax
from jax.experimental import pallas as pl
from jax.experimental.pallas import tpu as pltpu
from jax.experimental.pallas import tpu_sc as plsc

N_NODES = 10000
N_EDGES = 320000
H = 128
DEPTH = 3

NC = 2
NS = 16
NW = NC * NS
EPW = N_EDGES // NW
CH = 80
NCHUNK = EPW // CH
N_PAD = 10112
RPS = N_PAD // NS

EBLK = 2000
NBLK = 2000

_mesh = plsc.VectorSubcoreMesh(core_axis_name="c", subcore_axis_name="s")


K = 5


def _make_gather(n_edges, ch):
    epw = n_edges // NW
    n_outer = epw // ch // K

    @functools.partial(
        pl.kernel,
        out_type=jax.ShapeDtypeStruct((n_edges, H), jnp.float32),
        mesh=_mesh,
        scratch_types=[
            pltpu.VMEM((epw,), jnp.int32),
            pltpu.VMEM((epw,), jnp.int32),
            pltpu.VMEM((K, ch, H), jnp.float32),
            pltpu.SemaphoreType.DMA,
            pltpu.SemaphoreType.DMA,
            pltpu.SemaphoreType.DMA,
        ],
    )
    def gather(xni_hbm, xno_hbm, row_hbm, col_hbm, out_hbm,
               ridx_v, cidx_v, bufs, sem_g, sem_a, sem_w):
        wid = lax.axis_index("s") * NC + lax.axis_index("c")
        base = wid * epw
        pltpu.sync_copy(row_hbm.at[pl.ds(base, epw)], ridx_v)
        pltpu.sync_copy(col_hbm.at[pl.ds(base, epw)], cidx_v)

        def outer(g, _):
            @pl.when(g > 0)
            def _():
                for b in range(K):
                    off = base + ((g - 1) * K + b) * ch
                    pltpu.make_async_copy(
                        bufs.at[b], out_hbm.at[pl.ds(off, ch)], sem_w).wait()
            ds = []
            for b in range(K):
                c = (g * K + b) * ch
                ds.append(pltpu.async_copy(
                    xni_hbm.at[ridx_v.at[pl.ds(c, ch)]], bufs.at[b], sem_g))
            for d in ds:
                d.wait()
            ds = []
            for b in range(K):
                c = (g * K + b) * ch
                ds.append(pltpu.async_copy(
                    xno_hbm.at[cidx_v.at[pl.ds(c, ch)]], bufs.at[b], sem_a,
                    add=True))
            for d in ds:
                d.wait()
            for b in range(K):
                off = base + (g * K + b) * ch
                pltpu.async_copy(bufs.at[b], out_hbm.at[pl.ds(off, ch)],
                                 sem_w)
            return 0

        lax.fori_loop(0, n_outer, outer, 0)
        for b in range(K):
            off = base + ((n_outer - 1) * K + b) * ch
            pltpu.make_async_copy(
                bufs.at[b], out_hbm.at[pl.ds(off, ch)], sem_w).wait()

    return gather


def _make_scatter(n_edges, ch):
    epw = n_edges // NW
    n_outer = epw // ch // K

    @functools.partial(
        pl.kernel,
        out_type=jax.ShapeDtypeStruct((NC, N_PAD, H), jnp.float32),
        mesh=_mesh,
        scratch_types=[
            pltpu.VMEM((K, ch), jnp.int32),
            pltpu.VMEM((K, ch, H), jnp.float32),
            pltpu.VMEM_SHARED((N_PAD, H), jnp.float32),
            pltpu.SemaphoreType.DMA,
            pltpu.SemaphoreType.DMA,
            pltpu.SemaphoreType.DMA,
        ],
    )
    def scatter(m_hbm, col_hbm, zero_hbm, out_hbm, cidx2, bufs, acc_sh,
                sem_i, sem_m, sem_s):
        cid = lax.axis_index("c")
        sid = lax.axis_index("s")
        wid = sid * NC + cid
        pltpu.sync_copy(zero_hbm.at[pl.ds(sid * RPS, RPS)],
                        acc_sh.at[pl.ds(sid * RPS, RPS)])
        plsc.subcore_barrier()
        base = wid * epw

        def outer(g, _):
            @pl.when(g > 0)
            def _():
                for b in range(K):
                    pltpu.make_async_copy(
                        bufs.at[b], acc_sh.at[cidx2.at[b]], sem_s).wait()
            ds = []
            for b in range(K):
                off = base + (g * K + b) * ch
                ds.append(pltpu.async_copy(
                    col_hbm.at[pl.ds(off, ch)], cidx2.at[b], sem_i))
                ds.append(pltpu.async_copy(
                    m_hbm.at[pl.ds(off, ch)], bufs.at[b], sem_m))
            for d in ds:
                d.wait()
            for b in range(K):
                pltpu.async_copy(bufs.at[b], acc_sh.at[cidx2.at[b]], sem_s,
                                 add=True)
            return 0

        lax.fori_loop(0, n_outer, outer, 0)
        for b in range(K):
            pltpu.make_async_copy(bufs.at[b], acc_sh.at[cidx2.at[b]],
                                  sem_s).wait()
        plsc.subcore_barrier()
        pltpu.sync_copy(acc_sh.at[pl.ds(sid * RPS, RPS)],
                        out_hbm.at[cid, pl.ds(sid * RPS, RPS)])

    return scatter


E_A = 192000
E_B = 128000
_sc_gather_a = _make_gather(E_A, 120)
_sc_gather_b = _make_gather(E_B, 80)
_sc_scatter_a = _make_scatter(E_A, 40)
_sc_scatter_b = _make_scatter(E_B, 40)


def _dot(a, b):
    return jnp.dot(a, b, preferred_element_type=jnp.float32)


def _edge_body(scale_ref, edge_ref, fsum_ref, We_ref, eW1_ref, eW2_ref,
               n1W1_ref, n1W2_ref, be_ref, eb1_ref, eb2_ref, n1b1_ref,
               n1b2_ref, newe_ref, m_ref):
    e = edge_ref[...]
    h = jnp.maximum(_dot(e, We_ref[...]) + be_ref[...] + fsum_ref[...], 0.0)
    t = jnp.maximum(_dot(h, eW1_ref[...]) + eb1_ref[...], 0.0)
    ne = scale_ref[0] * e + _dot(t, eW2_ref[...]) + eb2_ref[...]
    u = jnp.maximum(_dot(ne, n1W1_ref[...]) + n1b1_ref[...], 0.0)
    newe_ref[...] = ne
    m_ref[...] = _dot(u, n1W2_ref[...]) + n1b2_ref[...]


_W_SPEC = pl.BlockSpec((H, H), lambda i: (0, 0))
_B_SPEC = pl.BlockSpec((1, H), lambda i: (0, 0))
_S_SPEC = pl.BlockSpec(memory_space=pltpu.SMEM)


def _make_edge_call(n_sub, blk_off):
    nblk = n_sub // EBLK
    off = blk_off

    return pl.pallas_call(
        _edge_body,
        grid=(nblk,),
        in_specs=[
            _S_SPEC,
            pl.BlockSpec((EBLK, H), lambda i: (i + off, 0)),
            pl.BlockSpec((EBLK, H), lambda i: (i, 0)),
            _W_SPEC, _W_SPEC, _W_SPEC, _W_SPEC, _W_SPEC,
            _B_SPEC, _B_SPEC, _B_SPEC, _B_SPEC, _B_SPEC,
        ],
        out_specs=[
            pl.BlockSpec((EBLK, H), lambda i: (i + off, 0)),
            pl.BlockSpec((EBLK, H), lambda i: (i, 0)),
        ],
        out_shape=[
            jax.ShapeDtypeStruct((N_EDGES, H), jnp.float32),
            jax.ShapeDtypeStruct((n_sub, H), jnp.float32),
        ],
        input_output_aliases={1: 0},
    )


_edge_call_a = _make_edge_call(E_A, 0)
_edge_call_b = _make_edge_call(E_B, E_A // EBLK)


def _node_body_proj(scale_ref, agga_ref, aggb_ref, x_ref, n2W1_ref, n2W2_ref,
                    Wni_ref, Wno_ref, n2b1_ref, n2b2_ref, xnew_ref, xni_ref,
                    xno_ref):
    agg = agga_ref[0] + agga_ref[1] + aggb_ref[0] + aggb_ref[1]
    t = jnp.maximum(_dot(agg, n2W1_ref[...]) + n2b1_ref[...], 0.0)
    xn = scale_ref[0] * x_ref[...] + _dot(t, n2W2_ref[...]) + n2b2_ref[...]
    xnew_ref[...] = xn
    xni_ref[...] = _dot(xn, Wni_ref[...])
    xno_ref[...] = _dot(xn, Wno_ref[...])


def _node_body_last(scale_ref, agga_ref, aggb_ref, x_ref, n2W1_ref, n2W2_ref,
                    n2b1_ref, n2b2_ref, xnew_ref):
    agg = agga_ref[0] + agga_ref[1] + aggb_ref[0] + aggb_ref[1]
    t = jnp.maximum(_dot(agg, n2W1_ref[...]) + n2b1_ref[...], 0.0)
    xnew_ref[...] = scale_ref[0] * x_ref[...] + _dot(t, n2W2_ref[...]) \
        + n2b2_ref[...]


_AGG_SPEC = pl.BlockSpec((NC, NBLK, H), lambda i: (0, i, 0))
_N_SPEC = pl.BlockSpec((NBLK, H), lambda i: (i, 0))
_NODE_SHAPE = jax.ShapeDtypeStruct((N_NODES, H), jnp.float32)

_node_call_proj = pl.pallas_call(
    _node_body_proj,
    grid=(N_NODES // NBLK,),
    in_specs=[_S_SPEC, _AGG_SPEC, _AGG_SPEC, _N_SPEC,
              _W_SPEC, _W_SPEC, _W_SPEC, _W_SPEC, _B_SPEC, _B_SPEC],
    out_specs=[_N_SPEC, _N_SPEC, _N_SPEC],
    out_shape=[_NODE_SHAPE, _NODE_SHAPE, _NODE_SHAPE],
)

_node_call_last = pl.pallas_call(
    _node_body_last,
    grid=(N_NODES // NBLK,),
    in_specs=[_S_SPEC, _AGG_SPEC, _AGG_SPEC, _N_SPEC, _W_SPEC, _W_SPEC,
              _B_SPEC, _B_SPEC],
    out_specs=_N_SPEC,
    out_shape=_NODE_SHAPE,
)


def _proj_body(x_ref, Wni_ref, Wno_ref, xni_ref, xno_ref):
    x = x_ref[...]
    xni_ref[...] = _dot(x, Wni_ref[...])
    xno_ref[...] = _dot(x, Wno_ref[...])


_proj_call = pl.pallas_call(
    _proj_body,
    grid=(N_NODES // NBLK,),
    in_specs=[_N_SPEC, _W_SPEC, _W_SPEC],
    out_specs=[_N_SPEC, _N_SPEC],
    out_shape=[_NODE_SHAPE, _NODE_SHAPE],
)


def kernel(x, edge_index, edge_attr, We, Wni, Wno, eW1, eW2, n1W1, n1W2,
           n2W1, n2W2, be, eb1, eb2, n1b1, n1b2, n2b1, n2b2, eps_e, eps_n):
    row_a, col_a = edge_index[0, :E_A], edge_index[1, :E_A]
    row_b, col_b = edge_index[0, E_A:], edge_index[1, E_A:]
    zeros_nh = jnp.zeros((N_PAD, H), jnp.float32)

    xni, xno = _proj_call(x, Wni[0], Wno[0])
    for i in range(DEPTH):
        scale_e = (1.0 + eps_e[i]).reshape((1,))
        wargs = (We[i], eW1[i], eW2[i], n1W1[i], n1W2[i],
                 be[i].reshape(1, H), eb1[i].reshape(1, H),
                 eb2[i].reshape(1, H), n1b1[i].reshape(1, H),
                 n1b2[i].reshape(1, H))
        fs_a = _sc_gather_a(xni, xno, row_a, col_a)
        fs_b = _sc_gather_b(xni, xno, row_b, col_b)
        edge_attr, m_a = _edge_call_a(scale_e, edge_attr, fs_a, *wargs)
        agg_a = _sc_scatter_a(m_a, col_a, zeros_nh)
        edge_attr, m_b = _edge_call_b(scale_e, edge_attr, fs_b, *wargs)
        agg_b = _sc_scatter_b(m_b, col_b, zeros_nh)
        scale_n = (1.0 + eps_n[i]).reshape((1,))
        if i + 1 < DEPTH:
            x, xni, xno = _node_call_proj(
                scale_n, agg_a, agg_b, x, n2W1[i], n2W2[i], Wni[i + 1],
                Wno[i + 1], n2b1[i].reshape(1, H), n2b2[i].reshape(1, H))
        else:
            x = _node_call_last(
                scale_n, agg_a, agg_b, x, n2W1[i], n2W2[i],
                n2b1[i].reshape(1, H), n2b2[i].reshape(1, H))
    return (x, edge_attr)


# speedup vs baseline: 1.1257x; 1.0002x over previous
import functools

import jax
import jax.numpy as jnp
from jax import lax
from jax.experimental import pallas as pl
from jax.experimental.pallas import tpu as pltpu
from jax.experimental.pallas import tpu_sc as plsc

N_NODES = 10000
N_EDGES = 320000
H = 128
DEPTH = 3

NC = 2
NS = 16
NW = NC * NS
EPW = N_EDGES // NW
CH = 80
NCHUNK = EPW // CH
N_PAD = 10112
RPS = N_PAD // NS

EBLK = 2000
NBLK = 2000

_mesh = plsc.VectorSubcoreMesh(core_axis_name="c", subcore_axis_name="s")


K = 5


def _make_gather(n_edges, ch):
    epw = n_edges // NW
    n_outer = epw // ch // K

    @functools.partial(
        pl.kernel,
        out_type=jax.ShapeDtypeStruct((n_edges, H), jnp.float32),
        mesh=_mesh,
        scratch_types=[
            pltpu.VMEM((epw,), jnp.int32),
            pltpu.VMEM((epw,), jnp.int32),
            pltpu.VMEM((K, ch, H), jnp.float32),
            pltpu.SemaphoreType.DMA,
            pltpu.SemaphoreType.DMA,
            pltpu.SemaphoreType.DMA,
        ],
    )
    def gather(xni_hbm, xno_hbm, row_hbm, col_hbm, out_hbm,
               ridx_v, cidx_v, bufs, sem_g, sem_a, sem_w):
        wid = lax.axis_index("s") * NC + lax.axis_index("c")
        base = wid * epw
        pltpu.sync_copy(row_hbm.at[pl.ds(base, epw)], ridx_v)
        pltpu.sync_copy(col_hbm.at[pl.ds(base, epw)], cidx_v)

        def outer(g, _):
            @pl.when(g > 0)
            def _():
                for b in range(K):
                    off = base + ((g - 1) * K + b) * ch
                    pltpu.make_async_copy(
                        bufs.at[b], out_hbm.at[pl.ds(off, ch)], sem_w).wait()
            ds = []
            for b in range(K):
                c = (g * K + b) * ch
                ds.append(pltpu.async_copy(
                    xni_hbm.at[ridx_v.at[pl.ds(c, ch)]], bufs.at[b], sem_g))
            for d in ds:
                d.wait()
            ds = []
            for b in range(K):
                c = (g * K + b) * ch
                ds.append(pltpu.async_copy(
                    xno_hbm.at[cidx_v.at[pl.ds(c, ch)]], bufs.at[b], sem_a,
                    add=True))
            for d in ds:
                d.wait()
            for b in range(K):
                off = base + (g * K + b) * ch
                pltpu.async_copy(bufs.at[b], out_hbm.at[pl.ds(off, ch)],
                                 sem_w)
            return 0

        lax.fori_loop(0, n_outer, outer, 0)
        for b in range(K):
            off = base + ((n_outer - 1) * K + b) * ch
            pltpu.make_async_copy(
                bufs.at[b], out_hbm.at[pl.ds(off, ch)], sem_w).wait()

    return gather


def _make_scatter(n_edges, ch, k):
    epw = n_edges // NW
    n_chunks = epw // ch
    n_outer = n_chunks // k
    rem = n_chunks - n_outer * k

    @functools.partial(
        pl.kernel,
        out_type=jax.ShapeDtypeStruct((NC, N_PAD, H), jnp.float32),
        mesh=_mesh,
        scratch_types=[
            pltpu.VMEM((k, ch), jnp.int32),
            pltpu.VMEM((k, ch, H), jnp.float32),
            pltpu.VMEM_SHARED((N_PAD, H), jnp.float32),
            pltpu.SemaphoreType.DMA,
            pltpu.SemaphoreType.DMA,
            pltpu.SemaphoreType.DMA,
        ],
    )
    def scatter(m_hbm, col_hbm, zero_hbm, out_hbm, cidx2, bufs, acc_sh,
                sem_i, sem_m, sem_s):
        cid = lax.axis_index("c")
        sid = lax.axis_index("s")
        wid = sid * NC + cid
        pltpu.sync_copy(zero_hbm.at[pl.ds(sid * RPS, RPS)],
                        acc_sh.at[pl.ds(sid * RPS, RPS)])
        plsc.subcore_barrier()
        base = wid * epw

        def batch(first_chunk, nb):
            ds = []
            for b in range(nb):
                off = base + first_chunk * ch + b * ch
                ds.append(pltpu.async_copy(
                    col_hbm.at[pl.ds(off, ch)], cidx2.at[b], sem_i))
                ds.append(pltpu.async_copy(
                    m_hbm.at[pl.ds(off, ch)], bufs.at[b], sem_m))
            for d in ds:
                d.wait()
            for b in range(nb):
                pltpu.async_copy(bufs.at[b], acc_sh.at[cidx2.at[b]], sem_s,
                                 add=True)

        def drain(nb):
            for b in range(nb):
                pltpu.make_async_copy(
                    bufs.at[b], acc_sh.at[cidx2.at[b]], sem_s).wait()

        def outer(g, _):
            @pl.when(g > 0)
            def _():
                drain(k)
            batch(g * k, k)
            return 0

        lax.fori_loop(0, n_outer, outer, 0)
        drain(k)
        if rem:
            batch(n_outer * k, rem)
            drain(rem)
        plsc.subcore_barrier()
        pltpu.sync_copy(acc_sh.at[pl.ds(sid * RPS, RPS)],
                        out_hbm.at[cid, pl.ds(sid * RPS, RPS)])

    return scatter


E_A = 192000
E_B = 128000
_sc_gather_a = _make_gather(E_A, 120)
_sc_gather_b = _make_gather(E_B, 80)
_sc_scatter_a = _make_scatter(E_A, 80, 4)
_sc_scatter_b = _make_scatter(E_B, 80, 4)


def _dot(a, b):
    return jnp.dot(a, b, preferred_element_type=jnp.float32)


def _edge_body(scale_ref, edge_ref, fsum_ref, We_ref, eW1_ref, eW2_ref,
               n1W1_ref, n1W2_ref, be_ref, eb1_ref, eb2_ref, n1b1_ref,
               n1b2_ref, newe_ref, m_ref):
    e = edge_ref[...]
    h = jnp.maximum(_dot(e, We_ref[...]) + be_ref[...] + fsum_ref[...], 0.0)
    t = jnp.maximum(_dot(h, eW1_ref[...]) + eb1_ref[...], 0.0)
    ne = scale_ref[0] * e + _dot(t, eW2_ref[...]) + eb2_ref[...]
    u = jnp.maximum(_dot(ne, n1W1_ref[...]) + n1b1_ref[...], 0.0)
    newe_ref[...] = ne
    m_ref[...] = _dot(u, n1W2_ref[...]) + n1b2_ref[...]


_W_SPEC = pl.BlockSpec((H, H), lambda i: (0, 0))
_B_SPEC = pl.BlockSpec((1, H), lambda i: (0, 0))
_S_SPEC = pl.BlockSpec(memory_space=pltpu.SMEM)


def _make_edge_call(n_sub, blk_off):
    nblk = n_sub // EBLK
    off = blk_off

    return pl.pallas_call(
        _edge_body,
        grid=(nblk,),
        in_specs=[
            _S_SPEC,
            pl.BlockSpec((EBLK, H), lambda i: (i + off, 0)),
            pl.BlockSpec((EBLK, H), lambda i: (i, 0)),
            _W_SPEC, _W_SPEC, _W_SPEC, _W_SPEC, _W_SPEC,
            _B_SPEC, _B_SPEC, _B_SPEC, _B_SPEC, _B_SPEC,
        ],
        out_specs=[
            pl.BlockSpec((EBLK, H), lambda i: (i + off, 0)),
            pl.BlockSpec((EBLK, H), lambda i: (i, 0)),
        ],
        out_shape=[
            jax.ShapeDtypeStruct((N_EDGES, H), jnp.float32),
            jax.ShapeDtypeStruct((n_sub, H), jnp.float32),
        ],
        input_output_aliases={1: 0},
    )


_edge_call_a = _make_edge_call(E_A, 0)
_edge_call_b = _make_edge_call(E_B, E_A // EBLK)


def _node_body_proj(scale_ref, agga_ref, aggb_ref, x_ref, n2W1_ref, n2W2_ref,
                    Wni_ref, Wno_ref, n2b1_ref, n2b2_ref, xnew_ref, xni_ref,
                    xno_ref):
    agg = agga_ref[0] + agga_ref[1] + aggb_ref[0] + aggb_ref[1]
    t = jnp.maximum(_dot(agg, n2W1_ref[...]) + n2b1_ref[...], 0.0)
    xn = scale_ref[0] * x_ref[...] + _dot(t, n2W2_ref[...]) + n2b2_ref[...]
    xnew_ref[...] = xn
    xni_ref[...] = _dot(xn, Wni_ref[...])
    xno_ref[...] = _dot(xn, Wno_ref[...])


def _node_body_last(scale_ref, agga_ref, aggb_ref, x_ref, n2W1_ref, n2W2_ref,
                    n2b1_ref, n2b2_ref, xnew_ref):
    agg = agga_ref[0] + agga_ref[1] + aggb_ref[0] + aggb_ref[1]
    t = jnp.maximum(_dot(agg, n2W1_ref[...]) + n2b1_ref[...], 0.0)
    xnew_ref[...] = scale_ref[0] * x_ref[...] + _dot(t, n2W2_ref[...]) \
        + n2b2_ref[...]


_AGG_SPEC = pl.BlockSpec((NC, NBLK, H), lambda i: (0, i, 0))
_N_SPEC = pl.BlockSpec((NBLK, H), lambda i: (i, 0))
_NODE_SHAPE = jax.ShapeDtypeStruct((N_NODES, H), jnp.float32)

_node_call_proj = pl.pallas_call(
    _node_body_proj,
    grid=(N_NODES // NBLK,),
    in_specs=[_S_SPEC, _AGG_SPEC, _AGG_SPEC, _N_SPEC,
              _W_SPEC, _W_SPEC, _W_SPEC, _W_SPEC, _B_SPEC, _B_SPEC],
    out_specs=[_N_SPEC, _N_SPEC, _N_SPEC],
    out_shape=[_NODE_SHAPE, _NODE_SHAPE, _NODE_SHAPE],
)

_node_call_last = pl.pallas_call(
    _node_body_last,
    grid=(N_NODES // NBLK,),
    in_specs=[_S_SPEC, _AGG_SPEC, _AGG_SPEC, _N_SPEC, _W_SPEC, _W_SPEC,
              _B_SPEC, _B_SPEC],
    out_specs=_N_SPEC,
    out_shape=_NODE_SHAPE,
)


def _proj_body(x_ref, Wni_ref, Wno_ref, xni_ref, xno_ref):
    x = x_ref[...]
    xni_ref[...] = _dot(x, Wni_ref[...])
    xno_ref[...] = _dot(x, Wno_ref[...])


_proj_call = pl.pallas_call(
    _proj_body,
    grid=(N_NODES // NBLK,),
    in_specs=[_N_SPEC, _W_SPEC, _W_SPEC],
    out_specs=[_N_SPEC, _N_SPEC],
    out_shape=[_NODE_SHAPE, _NODE_SHAPE],
)


def kernel(x, edge_index, edge_attr, We, Wni, Wno, eW1, eW2, n1W1, n1W2,
           n2W1, n2W2, be, eb1, eb2, n1b1, n1b2, n2b1, n2b2, eps_e, eps_n):
    row_a, col_a = edge_index[0, :E_A], edge_index[1, :E_A]
    row_b, col_b = edge_index[0, E_A:], edge_index[1, E_A:]
    zeros_nh = jnp.zeros((N_PAD, H), jnp.float32)

    xni, xno = _proj_call(x, Wni[0], Wno[0])
    for i in range(DEPTH):
        scale_e = (1.0 + eps_e[i]).reshape((1,))
        wargs = (We[i], eW1[i], eW2[i], n1W1[i], n1W2[i],
                 be[i].reshape(1, H), eb1[i].reshape(1, H),
                 eb2[i].reshape(1, H), n1b1[i].reshape(1, H),
                 n1b2[i].reshape(1, H))
        fs_a = _sc_gather_a(xni, xno, row_a, col_a)
        fs_b = _sc_gather_b(xni, xno, row_b, col_b)
        edge_attr, m_a = _edge_call_a(scale_e, edge_attr, fs_a, *wargs)
        agg_a = _sc_scatter_a(m_a, col_a, zeros_nh)
        edge_attr, m_b = _edge_call_b(scale_e, edge_attr, fs_b, *wargs)
        agg_b = _sc_scatter_b(m_b, col_b, zeros_nh)
        scale_n = (1.0 + eps_n[i]).reshape((1,))
        if i + 1 < DEPTH:
            x, xni, xno = _node_call_proj(
                scale_n, agg_a, agg_b, x, n2W1[i], n2W2[i], Wni[i + 1],
                Wno[i + 1], n2b1[i].reshape(1, H), n2b2[i].reshape(1, H))
        else:
            x = _node_call_last(
                scale_n, agg_a, agg_b, x, n2W1[i], n2W2[i],
                n2b1[i].reshape(1, H), n2b2[i].reshape(1, H))
    return (x, edge_attr)


# speedup vs baseline: 1.1279x; 1.0019x over previous
import functools

import jax
import jax.numpy as jnp
from jax import lax
from jax.experimental import pallas as pl
from jax.experimental.pallas import tpu as pltpu
from jax.experimental.pallas import tpu_sc as plsc

N_NODES = 10000
N_EDGES = 320000
H = 128
DEPTH = 3

NC = 2
NS = 16
NW = NC * NS
EPW = N_EDGES // NW
CH = 80
NCHUNK = EPW // CH
N_PAD = 10112
RPS = N_PAD // NS

EBLK = 2000
NBLK = 2000

_mesh = plsc.VectorSubcoreMesh(core_axis_name="c", subcore_axis_name="s")


K = 5


def _make_gather(n_edges, ch):
    epw = n_edges // NW
    n_outer = epw // ch // K

    @functools.partial(
        pl.kernel,
        out_type=jax.ShapeDtypeStruct((n_edges, H), jnp.float32),
        mesh=_mesh,
        scratch_types=[
            pltpu.VMEM((epw,), jnp.int32),
            pltpu.VMEM((epw,), jnp.int32),
            pltpu.VMEM((K, ch, H), jnp.float32),
            pltpu.SemaphoreType.DMA,
            pltpu.SemaphoreType.DMA,
            pltpu.SemaphoreType.DMA,
        ],
    )
    def gather(xni_hbm, xno_hbm, row_hbm, col_hbm, out_hbm,
               ridx_v, cidx_v, bufs, sem_g, sem_a, sem_w):
        wid = lax.axis_index("s") * NC + lax.axis_index("c")
        base = wid * epw
        pltpu.sync_copy(row_hbm.at[pl.ds(base, epw)], ridx_v)
        pltpu.sync_copy(col_hbm.at[pl.ds(base, epw)], cidx_v)

        def outer(g, _):
            @pl.when(g > 0)
            def _():
                for b in range(K):
                    off = base + ((g - 1) * K + b) * ch
                    pltpu.make_async_copy(
                        bufs.at[b], out_hbm.at[pl.ds(off, ch)], sem_w).wait()
            ds = []
            for b in range(K):
                c = (g * K + b) * ch
                ds.append(pltpu.async_copy(
                    xni_hbm.at[ridx_v.at[pl.ds(c, ch)]], bufs.at[b], sem_g))
            for d in ds:
                d.wait()
            ds = []
            for b in range(K):
                c = (g * K + b) * ch
                ds.append(pltpu.async_copy(
                    xno_hbm.at[cidx_v.at[pl.ds(c, ch)]], bufs.at[b], sem_a,
                    add=True))
            for d in ds:
                d.wait()
            for b in range(K):
                off = base + (g * K + b) * ch
                pltpu.async_copy(bufs.at[b], out_hbm.at[pl.ds(off, ch)],
                                 sem_w)
            return 0

        lax.fori_loop(0, n_outer, outer, 0)
        for b in range(K):
            off = base + ((n_outer - 1) * K + b) * ch
            pltpu.make_async_copy(
                bufs.at[b], out_hbm.at[pl.ds(off, ch)], sem_w).wait()

    return gather


def _make_scatter(n_edges, ch, k):
    epw = n_edges // NW
    n_chunks = epw // ch
    n_outer = n_chunks // k
    rem = n_chunks - n_outer * k

    @functools.partial(
        pl.kernel,
        out_type=jax.ShapeDtypeStruct((NC, N_PAD, H), jnp.float32),
        mesh=_mesh,
        scratch_types=(
            [pltpu.VMEM((ch,), jnp.int32)] * k
            + [pltpu.VMEM((ch, H), jnp.float32)] * k
            + [
                pltpu.VMEM_SHARED((N_PAD, H), jnp.float32),
                pltpu.SemaphoreType.DMA,
                pltpu.SemaphoreType.DMA,
                pltpu.SemaphoreType.DMA,
            ]
        ),
    )
    def scatter(m_hbm, col_hbm, zero_hbm, out_hbm, *rest):
        idxs = rest[:k]
        bufs = rest[k:2 * k]
        acc_sh, sem_i, sem_m, sem_s = rest[2 * k:]
        cid = lax.axis_index("c")
        sid = lax.axis_index("s")
        wid = sid * NC + cid
        pltpu.sync_copy(zero_hbm.at[pl.ds(sid * RPS, RPS)],
                        acc_sh.at[pl.ds(sid * RPS, RPS)])
        plsc.subcore_barrier()
        base = wid * epw

        def batch(first_chunk, nb):
            ds = []
            for b in range(nb):
                off = base + first_chunk * ch + b * ch
                ds.append(pltpu.async_copy(
                    col_hbm.at[pl.ds(off, ch)], idxs[b], sem_i))
                ds.append(pltpu.async_copy(
                    m_hbm.at[pl.ds(off, ch)], bufs[b], sem_m))
            for d in ds:
                d.wait()
            for b in range(nb):
                pltpu.async_copy(bufs[b], acc_sh.at[idxs[b]], sem_s,
                                 add=True)

        def drain(nb):
            for b in range(nb):
                pltpu.make_async_copy(
                    bufs[b], acc_sh.at[idxs[b]], sem_s).wait()

        def outer(g, _):
            @pl.when(g > 0)
            def _():
                drain(k)
            batch(g * k, k)
            return 0

        lax.fori_loop(0, n_outer, outer, 0)
        drain(k)
        if rem:
            batch(n_outer * k, rem)
            drain(rem)
        plsc.subcore_barrier()
        pltpu.sync_copy(acc_sh.at[pl.ds(sid * RPS, RPS)],
                        out_hbm.at[cid, pl.ds(sid * RPS, RPS)])

    return scatter


E_A = 192000
E_B = 128000
_sc_gather_a = _make_gather(E_A, 120)
_sc_gather_b = _make_gather(E_B, 80)
_sc_scatter_a = _make_scatter(E_A, 80, 4)
_sc_scatter_b = _make_scatter(E_B, 80, 4)


def _dot(a, b):
    return jnp.dot(a, b, preferred_element_type=jnp.float32)


def _edge_body(scale_ref, edge_ref, fsum_ref, We_ref, eW1_ref, eW2_ref,
               n1W1_ref, n1W2_ref, be_ref, eb1_ref, eb2_ref, n1b1_ref,
               n1b2_ref, newe_ref, m_ref):
    e = edge_ref[...]
    h = jnp.maximum(_dot(e, We_ref[...]) + be_ref[...] + fsum_ref[...], 0.0)
    t = jnp.maximum(_dot(h, eW1_ref[...]) + eb1_ref[...], 0.0)
    ne = scale_ref[0] * e + _dot(t, eW2_ref[...]) + eb2_ref[...]
    u = jnp.maximum(_dot(ne, n1W1_ref[...]) + n1b1_ref[...], 0.0)
    newe_ref[...] = ne
    m_ref[...] = _dot(u, n1W2_ref[...]) + n1b2_ref[...]


_W_SPEC = pl.BlockSpec((H, H), lambda i: (0, 0))
_B_SPEC = pl.BlockSpec((1, H), lambda i: (0, 0))
_S_SPEC = pl.BlockSpec(memory_space=pltpu.SMEM)


def _make_edge_call(n_sub, blk_off):
    nblk = n_sub // EBLK
    off = blk_off

    return pl.pallas_call(
        _edge_body,
        grid=(nblk,),
        in_specs=[
            _S_SPEC,
            pl.BlockSpec((EBLK, H), lambda i: (i + off, 0)),
            pl.BlockSpec((EBLK, H), lambda i: (i, 0)),
            _W_SPEC, _W_SPEC, _W_SPEC, _W_SPEC, _W_SPEC,
            _B_SPEC, _B_SPEC, _B_SPEC, _B_SPEC, _B_SPEC,
        ],
        out_specs=[
            pl.BlockSpec((EBLK, H), lambda i: (i + off, 0)),
            pl.BlockSpec((EBLK, H), lambda i: (i, 0)),
        ],
        out_shape=[
            jax.ShapeDtypeStruct((N_EDGES, H), jnp.float32),
            jax.ShapeDtypeStruct((n_sub, H), jnp.float32),
        ],
        input_output_aliases={1: 0},
    )


_edge_call_a = _make_edge_call(E_A, 0)
_edge_call_b = _make_edge_call(E_B, E_A // EBLK)


def _node_body_proj(scale_ref, agga_ref, aggb_ref, x_ref, n2W1_ref, n2W2_ref,
                    Wni_ref, Wno_ref, n2b1_ref, n2b2_ref, xnew_ref, xni_ref,
                    xno_ref):
    agg = agga_ref[0] + agga_ref[1] + aggb_ref[0] + aggb_ref[1]
    t = jnp.maximum(_dot(agg, n2W1_ref[...]) + n2b1_ref[...], 0.0)
    xn = scale_ref[0] * x_ref[...] + _dot(t, n2W2_ref[...]) + n2b2_ref[...]
    xnew_ref[...] = xn
    xni_ref[...] = _dot(xn, Wni_ref[...])
    xno_ref[...] = _dot(xn, Wno_ref[...])


def _node_body_last(scale_ref, agga_ref, aggb_ref, x_ref, n2W1_ref, n2W2_ref,
                    n2b1_ref, n2b2_ref, xnew_ref):
    agg = agga_ref[0] + agga_ref[1] + aggb_ref[0] + aggb_ref[1]
    t = jnp.maximum(_dot(agg, n2W1_ref[...]) + n2b1_ref[...], 0.0)
    xnew_ref[...] = scale_ref[0] * x_ref[...] + _dot(t, n2W2_ref[...]) \
        + n2b2_ref[...]


_AGG_SPEC = pl.BlockSpec((NC, NBLK, H), lambda i: (0, i, 0))
_N_SPEC = pl.BlockSpec((NBLK, H), lambda i: (i, 0))
_NODE_SHAPE = jax.ShapeDtypeStruct((N_NODES, H), jnp.float32)

_node_call_proj = pl.pallas_call(
    _node_body_proj,
    grid=(N_NODES // NBLK,),
    in_specs=[_S_SPEC, _AGG_SPEC, _AGG_SPEC, _N_SPEC,
              _W_SPEC, _W_SPEC, _W_SPEC, _W_SPEC, _B_SPEC, _B_SPEC],
    out_specs=[_N_SPEC, _N_SPEC, _N_SPEC],
    out_shape=[_NODE_SHAPE, _NODE_SHAPE, _NODE_SHAPE],
)

_node_call_last = pl.pallas_call(
    _node_body_last,
    grid=(N_NODES // NBLK,),
    in_specs=[_S_SPEC, _AGG_SPEC, _AGG_SPEC, _N_SPEC, _W_SPEC, _W_SPEC,
              _B_SPEC, _B_SPEC],
    out_specs=_N_SPEC,
    out_shape=_NODE_SHAPE,
)


def _proj_body(x_ref, Wni_ref, Wno_ref, xni_ref, xno_ref):
    x = x_ref[...]
    xni_ref[...] = _dot(x, Wni_ref[...])
    xno_ref[...] = _dot(x, Wno_ref[...])


_proj_call = pl.pallas_call(
    _proj_body,
    grid=(N_NODES // NBLK,),
    in_specs=[_N_SPEC, _W_SPEC, _W_SPEC],
    out_specs=[_N_SPEC, _N_SPEC],
    out_shape=[_NODE_SHAPE, _NODE_SHAPE],
)


def kernel(x, edge_index, edge_attr, We, Wni, Wno, eW1, eW2, n1W1, n1W2,
           n2W1, n2W2, be, eb1, eb2, n1b1, n1b2, n2b1, n2b2, eps_e, eps_n):
    row_a, col_a = edge_index[0, :E_A], edge_index[1, :E_A]
    row_b, col_b = edge_index[0, E_A:], edge_index[1, E_A:]
    zeros_nh = jnp.zeros((N_PAD, H), jnp.float32)

    xni, xno = _proj_call(x, Wni[0], Wno[0])
    for i in range(DEPTH):
        scale_e = (1.0 + eps_e[i]).reshape((1,))
        wargs = (We[i], eW1[i], eW2[i], n1W1[i], n1W2[i],
                 be[i].reshape(1, H), eb1[i].reshape(1, H),
                 eb2[i].reshape(1, H), n1b1[i].reshape(1, H),
                 n1b2[i].reshape(1, H))
        fs_a = _sc_gather_a(xni, xno, row_a, col_a)
        fs_b = _sc_gather_b(xni, xno, row_b, col_b)
        edge_attr, m_a = _edge_call_a(scale_e, edge_attr, fs_a, *wargs)
        agg_a = _sc_scatter_a(m_a, col_a, zeros_nh)
        edge_attr, m_b = _edge_call_b(scale_e, edge_attr, fs_b, *wargs)
        agg_b = _sc_scatter_b(m_b, col_b, zeros_nh)
        scale_n = (1.0 + eps_n[i]).reshape((1,))
        if i + 1 < DEPTH:
            x, xni, xno = _node_call_proj(
                scale_n, agg_a, agg_b, x, n2W1[i], n2W2[i], Wni[i + 1],
                Wno[i + 1], n2b1[i].reshape(1, H), n2b2[i].reshape(1, H))
        else:
            x = _node_call_last(
                scale_n, agg_a, agg_b, x, n2W1[i], n2W2[i],
                n2b1[i].reshape(1, H), n2b2[i].reshape(1, H))
    return (x, edge_attr)


# speedup vs baseline: 1.1549x; 1.0240x over previous
import functools

import jax
import jax.numpy as jnp
from jax import lax
from jax.experimental import pallas as pl
from jax.experimental.pallas import tpu as pltpu
from jax.experimental.pallas import tpu_sc as plsc

N_NODES = 10000
N_EDGES = 320000
H = 128
DEPTH = 3

NC = 2
NS = 16
NW = NC * NS
EPW = N_EDGES // NW
CH = 80
NCHUNK = EPW // CH
N_PAD = 10112
RPS = N_PAD // NS

EBLK = 2000
NBLK = 2000

_mesh = plsc.VectorSubcoreMesh(core_axis_name="c", subcore_axis_name="s")


K = 5


def _make_gather(n_edges, ch, k):
    epw = n_edges // NW
    n_chunks = epw // ch
    n_outer = n_chunks // k
    rem = n_chunks - n_outer * k

    @functools.partial(
        pl.kernel,
        out_type=jax.ShapeDtypeStruct((n_edges, H), jnp.float32),
        mesh=_mesh,
        scratch_types=[
            pltpu.VMEM((epw,), jnp.int32),
            pltpu.VMEM((epw,), jnp.int32),
            pltpu.VMEM((k, ch, H), jnp.float32),
            pltpu.SemaphoreType.DMA,
            pltpu.SemaphoreType.DMA,
            pltpu.SemaphoreType.DMA,
        ],
    )
    def gather(xni_hbm, xno_hbm, row_hbm, col_hbm, out_hbm,
               ridx_v, cidx_v, bufs, sem_g, sem_a, sem_w):
        wid = lax.axis_index("s") * NC + lax.axis_index("c")
        base = wid * epw
        pltpu.sync_copy(row_hbm.at[pl.ds(base, epw)], ridx_v)
        pltpu.sync_copy(col_hbm.at[pl.ds(base, epw)], cidx_v)

        def drain_wb(first_chunk, nb):
            for b in range(nb):
                off = base + (first_chunk + b) * ch
                pltpu.make_async_copy(
                    bufs.at[b], out_hbm.at[pl.ds(off, ch)], sem_w).wait()

        def batch(first_chunk, nb):
            ds = []
            for b in range(nb):
                c = (first_chunk + b) * ch
                ds.append(pltpu.async_copy(
                    xni_hbm.at[ridx_v.at[pl.ds(c, ch)]], bufs.at[b], sem_g))
            for d in ds:
                d.wait()
            ds = []
            for b in range(nb):
                c = (first_chunk + b) * ch
                ds.append(pltpu.async_copy(
                    xno_hbm.at[cidx_v.at[pl.ds(c, ch)]], bufs.at[b], sem_a,
                    add=True))
            for d in ds:
                d.wait()
            for b in range(nb):
                off = base + (first_chunk + b) * ch
                pltpu.async_copy(bufs.at[b], out_hbm.at[pl.ds(off, ch)],
                                 sem_w)

        def outer(g, _):
            @pl.when(g > 0)
            def _():
                drain_wb((g - 1) * k, k)
            batch(g * k, k)
            return 0

        lax.fori_loop(0, n_outer, outer, 0)
        drain_wb((n_outer - 1) * k, k)
        if rem:
            batch(n_outer * k, rem)
            drain_wb(n_outer * k, rem)

    return gather


def _make_scatter(n_edges, ch, k):
    epw = n_edges // NW
    n_chunks = epw // ch
    n_outer = n_chunks // k
    rem = n_chunks - n_outer * k

    @functools.partial(
        pl.kernel,
        out_type=jax.ShapeDtypeStruct((NC, N_PAD, H), jnp.float32),
        mesh=_mesh,
        scratch_types=(
            [pltpu.VMEM((ch,), jnp.int32)] * k
            + [pltpu.VMEM((ch, H), jnp.float32)] * k
            + [
                pltpu.VMEM_SHARED((N_PAD, H), jnp.float32),
                pltpu.SemaphoreType.DMA,
                pltpu.SemaphoreType.DMA,
                pltpu.SemaphoreType.DMA,
            ]
        ),
    )
    def scatter(m_hbm, col_hbm, zero_hbm, out_hbm, *rest):
        idxs = rest[:k]
        bufs = rest[k:2 * k]
        acc_sh, sem_i, sem_m, sem_s = rest[2 * k:]
        cid = lax.axis_index("c")
        sid = lax.axis_index("s")
        wid = sid * NC + cid
        pltpu.sync_copy(zero_hbm.at[pl.ds(sid * RPS, RPS)],
                        acc_sh.at[pl.ds(sid * RPS, RPS)])
        plsc.subcore_barrier()
        base = wid * epw

        def batch(first_chunk, nb):
            ds = []
            for b in range(nb):
                off = base + first_chunk * ch + b * ch
                ds.append(pltpu.async_copy(
                    col_hbm.at[pl.ds(off, ch)], idxs[b], sem_i))
                ds.append(pltpu.async_copy(
                    m_hbm.at[pl.ds(off, ch)], bufs[b], sem_m))
            for d in ds:
                d.wait()
            for b in range(nb):
                pltpu.async_copy(bufs[b], acc_sh.at[idxs[b]], sem_s,
                                 add=True)

        def drain(nb):
            for b in range(nb):
                pltpu.make_async_copy(
                    bufs[b], acc_sh.at[idxs[b]], sem_s).wait()

        def outer(g, _):
            @pl.when(g > 0)
            def _():
                drain(k)
            batch(g * k, k)
            return 0

        lax.fori_loop(0, n_outer, outer, 0)
        drain(k)
        if rem:
            batch(n_outer * k, rem)
            drain(rem)
        plsc.subcore_barrier()
        pltpu.sync_copy(acc_sh.at[pl.ds(sid * RPS, RPS)],
                        out_hbm.at[cid, pl.ds(sid * RPS, RPS)])

    return scatter


E_A = 128000
E_B = 96000
E_C = 96000
_sc_gather_a = _make_gather(E_A, 80, 6)
_sc_gather_b = _make_gather(E_B, 120, 5)
_sc_gather_c = _make_gather(E_C, 120, 5)
_sc_scatter_a = _make_scatter(E_A, 80, 4)
_sc_scatter_b = _make_scatter(E_B, 40, 8)
_sc_scatter_c = _make_scatter(E_C, 40, 8)


def _dot(a, b):
    return jnp.dot(a, b, preferred_element_type=jnp.float32)


def _edge_body(scale_ref, edge_ref, fsum_ref, We_ref, eW1_ref, eW2_ref,
               n1W1_ref, n1W2_ref, be_ref, eb1_ref, eb2_ref, n1b1_ref,
               n1b2_ref, newe_ref, m_ref):
    e = edge_ref[...]
    h = jnp.maximum(_dot(e, We_ref[...]) + be_ref[...] + fsum_ref[...], 0.0)
    t = jnp.maximum(_dot(h, eW1_ref[...]) + eb1_ref[...], 0.0)
    ne = scale_ref[0] * e + _dot(t, eW2_ref[...]) + eb2_ref[...]
    u = jnp.maximum(_dot(ne, n1W1_ref[...]) + n1b1_ref[...], 0.0)
    newe_ref[...] = ne
    m_ref[...] = _dot(u, n1W2_ref[...]) + n1b2_ref[...]


_W_SPEC = pl.BlockSpec((H, H), lambda i: (0, 0))
_B_SPEC = pl.BlockSpec((1, H), lambda i: (0, 0))
_S_SPEC = pl.BlockSpec(memory_space=pltpu.SMEM)


def _make_edge_call(n_sub, blk_off):
    nblk = n_sub // EBLK
    off = blk_off

    return pl.pallas_call(
        _edge_body,
        grid=(nblk,),
        in_specs=[
            _S_SPEC,
            pl.BlockSpec((EBLK, H), lambda i: (i + off, 0)),
            pl.BlockSpec((EBLK, H), lambda i: (i, 0)),
            _W_SPEC, _W_SPEC, _W_SPEC, _W_SPEC, _W_SPEC,
            _B_SPEC, _B_SPEC, _B_SPEC, _B_SPEC, _B_SPEC,
        ],
        out_specs=[
            pl.BlockSpec((EBLK, H), lambda i: (i + off, 0)),
            pl.BlockSpec((EBLK, H), lambda i: (i, 0)),
        ],
        out_shape=[
            jax.ShapeDtypeStruct((N_EDGES, H), jnp.float32),
            jax.ShapeDtypeStruct((n_sub, H), jnp.float32),
        ],
        input_output_aliases={1: 0},
    )


_edge_call_a = _make_edge_call(E_A, 0)
_edge_call_b = _make_edge_call(E_B, E_A // EBLK)
_edge_call_c = _make_edge_call(E_C, (E_A + E_B) // EBLK)


def _node_body_proj(scale_ref, agga_ref, aggb_ref, aggc_ref, x_ref, n2W1_ref,
                    n2W2_ref, Wni_ref, Wno_ref, n2b1_ref, n2b2_ref, xnew_ref,
                    xni_ref, xno_ref):
    agg = (agga_ref[0] + agga_ref[1] + aggb_ref[0] + aggb_ref[1]
           + aggc_ref[0] + aggc_ref[1])
    t = jnp.maximum(_dot(agg, n2W1_ref[...]) + n2b1_ref[...], 0.0)
    xn = scale_ref[0] * x_ref[...] + _dot(t, n2W2_ref[...]) + n2b2_ref[...]
    xnew_ref[...] = xn
    xni_ref[...] = _dot(xn, Wni_ref[...])
    xno_ref[...] = _dot(xn, Wno_ref[...])


def _node_body_last(scale_ref, agga_ref, aggb_ref, aggc_ref, x_ref, n2W1_ref,
                    n2W2_ref, n2b1_ref, n2b2_ref, xnew_ref):
    agg = (agga_ref[0] + agga_ref[1] + aggb_ref[0] + aggb_ref[1]
           + aggc_ref[0] + aggc_ref[1])
    t = jnp.maximum(_dot(agg, n2W1_ref[...]) + n2b1_ref[...], 0.0)
    xnew_ref[...] = scale_ref[0] * x_ref[...] + _dot(t, n2W2_ref[...]) \
        + n2b2_ref[...]


_AGG_SPEC = pl.BlockSpec((NC, NBLK, H), lambda i: (0, i, 0))
_N_SPEC = pl.BlockSpec((NBLK, H), lambda i: (i, 0))
_NODE_SHAPE = jax.ShapeDtypeStruct((N_NODES, H), jnp.float32)

_node_call_proj = pl.pallas_call(
    _node_body_proj,
    grid=(N_NODES // NBLK,),
    in_specs=[_S_SPEC, _AGG_SPEC, _AGG_SPEC, _AGG_SPEC, _N_SPEC,
              _W_SPEC, _W_SPEC, _W_SPEC, _W_SPEC, _B_SPEC, _B_SPEC],
    out_specs=[_N_SPEC, _N_SPEC, _N_SPEC],
    out_shape=[_NODE_SHAPE, _NODE_SHAPE, _NODE_SHAPE],
)

_node_call_last = pl.pallas_call(
    _node_body_last,
    grid=(N_NODES // NBLK,),
    in_specs=[_S_SPEC, _AGG_SPEC, _AGG_SPEC, _AGG_SPEC, _N_SPEC, _W_SPEC,
              _W_SPEC, _B_SPEC, _B_SPEC],
    out_specs=_N_SPEC,
    out_shape=_NODE_SHAPE,
)


def _proj_body(x_ref, Wni_ref, Wno_ref, xni_ref, xno_ref):
    x = x_ref[...]
    xni_ref[...] = _dot(x, Wni_ref[...])
    xno_ref[...] = _dot(x, Wno_ref[...])


_proj_call = pl.pallas_call(
    _proj_body,
    grid=(N_NODES // NBLK,),
    in_specs=[_N_SPEC, _W_SPEC, _W_SPEC],
    out_specs=[_N_SPEC, _N_SPEC],
    out_shape=[_NODE_SHAPE, _NODE_SHAPE],
)


def kernel(x, edge_index, edge_attr, We, Wni, Wno, eW1, eW2, n1W1, n1W2,
           n2W1, n2W2, be, eb1, eb2, n1b1, n1b2, n2b1, n2b2, eps_e, eps_n):
    row_a, col_a = edge_index[0, :E_A], edge_index[1, :E_A]
    row_b, col_b = edge_index[0, E_A:E_A + E_B], edge_index[1, E_A:E_A + E_B]
    row_c, col_c = edge_index[0, E_A + E_B:], edge_index[1, E_A + E_B:]
    zeros_nh = jnp.zeros((N_PAD, H), jnp.float32)

    xni, xno = _proj_call(x, Wni[0], Wno[0])
    for i in range(DEPTH):
        scale_e = (1.0 + eps_e[i]).reshape((1,))
        wargs = (We[i], eW1[i], eW2[i], n1W1[i], n1W2[i],
                 be[i].reshape(1, H), eb1[i].reshape(1, H),
                 eb2[i].reshape(1, H), n1b1[i].reshape(1, H),
                 n1b2[i].reshape(1, H))
        fs_a = _sc_gather_a(xni, xno, row_a, col_a)
        fs_b = _sc_gather_b(xni, xno, row_b, col_b)
        edge_attr, m_a = _edge_call_a(scale_e, edge_attr, fs_a, *wargs)
        fs_c = _sc_gather_c(xni, xno, row_c, col_c)
        edge_attr, m_b = _edge_call_b(scale_e, edge_attr, fs_b, *wargs)
        agg_a = _sc_scatter_a(m_a, col_a, zeros_nh)
        edge_attr, m_c = _edge_call_c(scale_e, edge_attr, fs_c, *wargs)
        agg_b = _sc_scatter_b(m_b, col_b, zeros_nh)
        agg_c = _sc_scatter_c(m_c, col_c, zeros_nh)
        scale_n = (1.0 + eps_n[i]).reshape((1,))
        if i + 1 < DEPTH:
            x, xni, xno = _node_call_proj(
                scale_n, agg_a, agg_b, agg_c, x, n2W1[i], n2W2[i],
                Wni[i + 1], Wno[i + 1], n2b1[i].reshape(1, H),
                n2b2[i].reshape(1, H))
        else:
            x = _node_call_last(
                scale_n, agg_a, agg_b, agg_c, x, n2W1[i], n2W2[i],
                n2b1[i].reshape(1, H), n2b2[i].reshape(1, H))
    return (x, edge_attr)


# speedup vs baseline: 1.1852x; 1.0262x over previous
import functools

import jax
import jax.numpy as jnp
from jax import lax
from jax.experimental import pallas as pl
from jax.experimental.pallas import tpu as pltpu
from jax.experimental.pallas import tpu_sc as plsc

N_NODES = 10000
N_EDGES = 320000
H = 128
DEPTH = 3

NC = 2
NS = 16
NW = NC * NS
EPW = N_EDGES // NW
CH = 80
NCHUNK = EPW // CH
N_PAD = 10112
RPS = N_PAD // NS

EBLK = 2000
NBLK = 2000

_mesh = plsc.VectorSubcoreMesh(core_axis_name="c", subcore_axis_name="s")


K = 5


def _make_gather(n_edges, ch, k):
    epw = n_edges // NW
    n_chunks = epw // ch
    n_outer = n_chunks // k
    rem = n_chunks - n_outer * k

    @functools.partial(
        pl.kernel,
        out_type=jax.ShapeDtypeStruct((n_edges, H), jnp.float32),
        mesh=_mesh,
        scratch_types=[
            pltpu.VMEM((epw,), jnp.int32),
            pltpu.VMEM((epw,), jnp.int32),
            pltpu.VMEM((k, ch, H), jnp.float32),
            pltpu.SemaphoreType.DMA,
            pltpu.SemaphoreType.DMA,
            pltpu.SemaphoreType.DMA,
        ],
    )
    def gather(xni_hbm, xno_hbm, row_hbm, col_hbm, out_hbm,
               ridx_v, cidx_v, bufs, sem_g, sem_a, sem_w):
        wid = lax.axis_index("s") * NC + lax.axis_index("c")
        base = wid * epw
        pltpu.sync_copy(row_hbm.at[pl.ds(base, epw)], ridx_v)
        pltpu.sync_copy(col_hbm.at[pl.ds(base, epw)], cidx_v)

        def drain_wb(first_chunk, nb):
            for b in range(nb):
                off = base + (first_chunk + b) * ch
                pltpu.make_async_copy(
                    bufs.at[b], out_hbm.at[pl.ds(off, ch)], sem_w).wait()

        def batch(first_chunk, nb):
            gds, ads = [], []
            for b in range(nb):
                c = (first_chunk + b) * ch
                gds.append(pltpu.async_copy(
                    xni_hbm.at[ridx_v.at[pl.ds(c, ch)]], bufs.at[b], sem_g))
            for b in range(nb):
                c = (first_chunk + b) * ch
                gds[b].wait()
                ads.append(pltpu.async_copy(
                    xno_hbm.at[cidx_v.at[pl.ds(c, ch)]], bufs.at[b], sem_a,
                    add=True))
            for b in range(nb):
                off = base + (first_chunk + b) * ch
                ads[b].wait()
                pltpu.async_copy(bufs.at[b], out_hbm.at[pl.ds(off, ch)],
                                 sem_w)

        def outer(g, _):
            @pl.when(g > 0)
            def _():
                drain_wb((g - 1) * k, k)
            batch(g * k, k)
            return 0

        lax.fori_loop(0, n_outer, outer, 0)
        drain_wb((n_outer - 1) * k, k)
        if rem:
            batch(n_outer * k, rem)
            drain_wb(n_outer * k, rem)

    return gather


def _make_scatter(n_edges, ch, k):
    epw = n_edges // NW
    n_chunks = epw // ch
    n_outer = n_chunks // k
    rem = n_chunks - n_outer * k

    @functools.partial(
        pl.kernel,
        out_type=jax.ShapeDtypeStruct((NC, N_PAD, H), jnp.float32),
        mesh=_mesh,
        scratch_types=(
            [pltpu.VMEM((ch,), jnp.int32)] * k
            + [pltpu.VMEM((ch, H), jnp.float32)] * k
            + [
                pltpu.VMEM_SHARED((N_PAD, H), jnp.float32),
                pltpu.SemaphoreType.DMA,
                pltpu.SemaphoreType.DMA,
                pltpu.SemaphoreType.DMA,
            ]
        ),
    )
    def scatter(m_hbm, col_hbm, zero_hbm, out_hbm, *rest):
        idxs = rest[:k]
        bufs = rest[k:2 * k]
        acc_sh, sem_i, sem_m, sem_s = rest[2 * k:]
        cid = lax.axis_index("c")
        sid = lax.axis_index("s")
        wid = sid * NC + cid
        pltpu.sync_copy(zero_hbm.at[pl.ds(sid * RPS, RPS)],
                        acc_sh.at[pl.ds(sid * RPS, RPS)])
        plsc.subcore_barrier()
        base = wid * epw

        def batch(first_chunk, nb):
            ids, mds = [], []
            for b in range(nb):
                off = base + first_chunk * ch + b * ch
                ids.append(pltpu.async_copy(
                    col_hbm.at[pl.ds(off, ch)], idxs[b], sem_i))
                mds.append(pltpu.async_copy(
                    m_hbm.at[pl.ds(off, ch)], bufs[b], sem_m))
            for b in range(nb):
                ids[b].wait()
                mds[b].wait()
                pltpu.async_copy(bufs[b], acc_sh.at[idxs[b]], sem_s,
                                 add=True)

        def drain(nb):
            for b in range(nb):
                pltpu.make_async_copy(
                    bufs[b], acc_sh.at[idxs[b]], sem_s).wait()

        def outer(g, _):
            @pl.when(g > 0)
            def _():
                drain(k)
            batch(g * k, k)
            return 0

        lax.fori_loop(0, n_outer, outer, 0)
        drain(k)
        if rem:
            batch(n_outer * k, rem)
            drain(rem)
        plsc.subcore_barrier()
        pltpu.sync_copy(acc_sh.at[pl.ds(sid * RPS, RPS)],
                        out_hbm.at[cid, pl.ds(sid * RPS, RPS)])

    return scatter


E_A = 128000
E_B = 96000
E_C = 96000
_sc_gather_a = _make_gather(E_A, 80, 10)
_sc_gather_b = _make_gather(E_B, 120, 5)
_sc_gather_c = _make_gather(E_C, 120, 5)
_sc_scatter_a = _make_scatter(E_A, 80, 4)
_sc_scatter_b = _make_scatter(E_B, 40, 8)
_sc_scatter_c = _make_scatter(E_C, 40, 8)


def _dot(a, b):
    return jnp.dot(a, b, preferred_element_type=jnp.float32)


def _edge_body(scale_ref, edge_ref, fsum_ref, We_ref, eW1_ref, eW2_ref,
               n1W1_ref, n1W2_ref, be_ref, eb1_ref, eb2_ref, n1b1_ref,
               n1b2_ref, newe_ref, m_ref):
    e = edge_ref[...]
    h = jnp.maximum(_dot(e, We_ref[...]) + be_ref[...] + fsum_ref[...], 0.0)
    t = jnp.maximum(_dot(h, eW1_ref[...]) + eb1_ref[...], 0.0)
    ne = scale_ref[0] * e + _dot(t, eW2_ref[...]) + eb2_ref[...]
    u = jnp.maximum(_dot(ne, n1W1_ref[...]) + n1b1_ref[...], 0.0)
    newe_ref[...] = ne
    m_ref[...] = _dot(u, n1W2_ref[...]) + n1b2_ref[...]


_W_SPEC = pl.BlockSpec((H, H), lambda i: (0, 0))
_B_SPEC = pl.BlockSpec((1, H), lambda i: (0, 0))
_S_SPEC = pl.BlockSpec(memory_space=pltpu.SMEM)


def _make_edge_call(n_sub, blk_off):
    nblk = n_sub // EBLK
    off = blk_off

    return pl.pallas_call(
        _edge_body,
        grid=(nblk,),
        in_specs=[
            _S_SPEC,
            pl.BlockSpec((EBLK, H), lambda i: (i + off, 0)),
            pl.BlockSpec((EBLK, H), lambda i: (i, 0)),
            _W_SPEC, _W_SPEC, _W_SPEC, _W_SPEC, _W_SPEC,
            _B_SPEC, _B_SPEC, _B_SPEC, _B_SPEC, _B_SPEC,
        ],
        out_specs=[
            pl.BlockSpec((EBLK, H), lambda i: (i + off, 0)),
            pl.BlockSpec((EBLK, H), lambda i: (i, 0)),
        ],
        out_shape=[
            jax.ShapeDtypeStruct((N_EDGES, H), jnp.float32),
            jax.ShapeDtypeStruct((n_sub, H), jnp.float32),
        ],
        input_output_aliases={1: 0},
    )


_edge_call_a = _make_edge_call(E_A, 0)
_edge_call_b = _make_edge_call(E_B, E_A // EBLK)
_edge_call_c = _make_edge_call(E_C, (E_A + E_B) // EBLK)


def _node_body_proj(scale_ref, agga_ref, aggb_ref, aggc_ref, x_ref, n2W1_ref,
                    n2W2_ref, Wni_ref, Wno_ref, n2b1_ref, n2b2_ref, xnew_ref,
                    xni_ref, xno_ref):
    agg = (agga_ref[0] + agga_ref[1] + aggb_ref[0] + aggb_ref[1]
           + aggc_ref[0] + aggc_ref[1])
    t = jnp.maximum(_dot(agg, n2W1_ref[...]) + n2b1_ref[...], 0.0)
    xn = scale_ref[0] * x_ref[...] + _dot(t, n2W2_ref[...]) + n2b2_ref[...]
    xnew_ref[...] = xn
    xni_ref[...] = _dot(xn, Wni_ref[...])
    xno_ref[...] = _dot(xn, Wno_ref[...])


def _node_body_last(scale_ref, agga_ref, aggb_ref, aggc_ref, x_ref, n2W1_ref,
                    n2W2_ref, n2b1_ref, n2b2_ref, xnew_ref):
    agg = (agga_ref[0] + agga_ref[1] + aggb_ref[0] + aggb_ref[1]
           + aggc_ref[0] + aggc_ref[1])
    t = jnp.maximum(_dot(agg, n2W1_ref[...]) + n2b1_ref[...], 0.0)
    xnew_ref[...] = scale_ref[0] * x_ref[...] + _dot(t, n2W2_ref[...]) \
        + n2b2_ref[...]


_AGG_SPEC = pl.BlockSpec((NC, NBLK, H), lambda i: (0, i, 0))
_N_SPEC = pl.BlockSpec((NBLK, H), lambda i: (i, 0))
_NODE_SHAPE = jax.ShapeDtypeStruct((N_NODES, H), jnp.float32)

_node_call_proj = pl.pallas_call(
    _node_body_proj,
    grid=(N_NODES // NBLK,),
    in_specs=[_S_SPEC, _AGG_SPEC, _AGG_SPEC, _AGG_SPEC, _N_SPEC,
              _W_SPEC, _W_SPEC, _W_SPEC, _W_SPEC, _B_SPEC, _B_SPEC],
    out_specs=[_N_SPEC, _N_SPEC, _N_SPEC],
    out_shape=[_NODE_SHAPE, _NODE_SHAPE, _NODE_SHAPE],
)

_node_call_last = pl.pallas_call(
    _node_body_last,
    grid=(N_NODES // NBLK,),
    in_specs=[_S_SPEC, _AGG_SPEC, _AGG_SPEC, _AGG_SPEC, _N_SPEC, _W_SPEC,
              _W_SPEC, _B_SPEC, _B_SPEC],
    out_specs=_N_SPEC,
    out_shape=_NODE_SHAPE,
)


def _proj_body(x_ref, Wni_ref, Wno_ref, xni_ref, xno_ref):
    x = x_ref[...]
    xni_ref[...] = _dot(x, Wni_ref[...])
    xno_ref[...] = _dot(x, Wno_ref[...])


_proj_call = pl.pallas_call(
    _proj_body,
    grid=(N_NODES // NBLK,),
    in_specs=[_N_SPEC, _W_SPEC, _W_SPEC],
    out_specs=[_N_SPEC, _N_SPEC],
    out_shape=[_NODE_SHAPE, _NODE_SHAPE],
)


def kernel(x, edge_index, edge_attr, We, Wni, Wno, eW1, eW2, n1W1, n1W2,
           n2W1, n2W2, be, eb1, eb2, n1b1, n1b2, n2b1, n2b2, eps_e, eps_n):
    row_a, col_a = edge_index[0, :E_A], edge_index[1, :E_A]
    row_b, col_b = edge_index[0, E_A:E_A + E_B], edge_index[1, E_A:E_A + E_B]
    row_c, col_c = edge_index[0, E_A + E_B:], edge_index[1, E_A + E_B:]
    zeros_nh = jnp.zeros((N_PAD, H), jnp.float32)

    xni, xno = _proj_call(x, Wni[0], Wno[0])
    for i in range(DEPTH):
        scale_e = (1.0 + eps_e[i]).reshape((1,))
        wargs = (We[i], eW1[i], eW2[i], n1W1[i], n1W2[i],
                 be[i].reshape(1, H), eb1[i].reshape(1, H),
                 eb2[i].reshape(1, H), n1b1[i].reshape(1, H),
                 n1b2[i].reshape(1, H))
        fs_a = _sc_gather_a(xni, xno, row_a, col_a)
        fs_b = _sc_gather_b(xni, xno, row_b, col_b)
        edge_attr, m_a = _edge_call_a(scale_e, edge_attr, fs_a, *wargs)
        fs_c = _sc_gather_c(xni, xno, row_c, col_c)
        edge_attr, m_b = _edge_call_b(scale_e, edge_attr, fs_b, *wargs)
        agg_a = _sc_scatter_a(m_a, col_a, zeros_nh)
        edge_attr, m_c = _edge_call_c(scale_e, edge_attr, fs_c, *wargs)
        agg_b = _sc_scatter_b(m_b, col_b, zeros_nh)
        agg_c = _sc_scatter_c(m_c, col_c, zeros_nh)
        scale_n = (1.0 + eps_n[i]).reshape((1,))
        if i + 1 < DEPTH:
            x, xni, xno = _node_call_proj(
                scale_n, agg_a, agg_b, agg_c, x, n2W1[i], n2W2[i],
                Wni[i + 1], Wno[i + 1], n2b1[i].reshape(1, H),
                n2b2[i].reshape(1, H))
        else:
            x = _node_call_last(
                scale_n, agg_a, agg_b, agg_c, x, n2W1[i], n2W2[i],
                n2b1[i].reshape(1, H), n2b2[i].reshape(1, H))
    return (x, edge_attr)


# speedup vs baseline: 1.3377x; 1.1286x over previous
import functools

import jax
import jax.numpy as jnp
from jax import lax
from jax.experimental import pallas as pl
from jax.experimental.pallas import tpu as pltpu
from jax.experimental.pallas import tpu_sc as plsc

N_NODES = 10000
N_EDGES = 320000
H = 128
DEPTH = 3

NC = 2
NS = 16
NW = NC * NS
EPW = N_EDGES // NW
CH = 80
NCHUNK = EPW // CH
N_PAD = 10112
RPS = N_PAD // NS

EBLK = 2000
NBLK = 2000

_mesh = plsc.VectorSubcoreMesh(core_axis_name="c", subcore_axis_name="s")


K = 5


def _make_gather(n_edges, ch, k):
    epw = n_edges // NW
    n_chunks = epw // ch
    n_outer = n_chunks // k
    rem = n_chunks - n_outer * k

    @functools.partial(
        pl.kernel,
        out_type=jax.ShapeDtypeStruct((n_edges, H), jnp.float32),
        mesh=_mesh,
        scratch_types=[
            pltpu.VMEM((epw,), jnp.int32),
            pltpu.VMEM((epw,), jnp.int32),
            pltpu.VMEM((k, ch, H), jnp.float32),
            pltpu.SemaphoreType.DMA,
            pltpu.SemaphoreType.DMA,
            pltpu.SemaphoreType.DMA,
        ],
    )
    def gather(xni_hbm, xno_hbm, row_hbm, col_hbm, out_hbm,
               ridx_v, cidx_v, bufs, sem_g, sem_a, sem_w):
        wid = lax.axis_index("s") * NC + lax.axis_index("c")
        base = wid * epw
        pltpu.sync_copy(row_hbm.at[pl.ds(base, epw)], ridx_v)
        pltpu.sync_copy(col_hbm.at[pl.ds(base, epw)], cidx_v)

        def drain_wb(first_chunk, nb):
            for b in range(nb):
                off = base + (first_chunk + b) * ch
                pltpu.make_async_copy(
                    bufs.at[b], out_hbm.at[pl.ds(off, ch)], sem_w).wait()

        def batch(first_chunk, nb):
            gds, ads = [], []
            for b in range(nb):
                c = (first_chunk + b) * ch
                gds.append(pltpu.async_copy(
                    xni_hbm.at[ridx_v.at[pl.ds(c, ch)]], bufs.at[b], sem_g))
            for b in range(nb):
                c = (first_chunk + b) * ch
                gds[b].wait()
                ads.append(pltpu.async_copy(
                    xno_hbm.at[cidx_v.at[pl.ds(c, ch)]], bufs.at[b], sem_a,
                    add=True))
            for b in range(nb):
                off = base + (first_chunk + b) * ch
                ads[b].wait()
                pltpu.async_copy(bufs.at[b], out_hbm.at[pl.ds(off, ch)],
                                 sem_w)

        def outer(g, _):
            @pl.when(g > 0)
            def _():
                drain_wb((g - 1) * k, k)
            batch(g * k, k)
            return 0

        lax.fori_loop(0, n_outer, outer, 0)
        drain_wb((n_outer - 1) * k, k)
        if rem:
            batch(n_outer * k, rem)
            drain_wb(n_outer * k, rem)

    return gather


def _make_scatter(n_edges, ch, k):
    epw = n_edges // NW
    n_chunks = epw // ch
    n_outer = n_chunks // k
    rem = n_chunks - n_outer * k

    @functools.partial(
        pl.kernel,
        out_type=jax.ShapeDtypeStruct((NC, N_PAD, H), jnp.float32),
        mesh=_mesh,
        scratch_types=(
            [pltpu.VMEM((ch,), jnp.int32)] * k
            + [pltpu.VMEM((ch, H), jnp.float32)] * k
            + [
                pltpu.VMEM_SHARED((N_PAD, H), jnp.float32),
                pltpu.SemaphoreType.DMA,
                pltpu.SemaphoreType.DMA,
                pltpu.SemaphoreType.DMA,
            ]
        ),
    )
    def scatter(m_hbm, col_hbm, zero_hbm, out_hbm, *rest):
        idxs = rest[:k]
        bufs = rest[k:2 * k]
        acc_sh, sem_i, sem_m, sem_s = rest[2 * k:]
        cid = lax.axis_index("c")
        sid = lax.axis_index("s")
        wid = sid * NC + cid
        pltpu.sync_copy(zero_hbm.at[pl.ds(sid * RPS, RPS)],
                        acc_sh.at[pl.ds(sid * RPS, RPS)])
        plsc.subcore_barrier()
        base = wid * epw

        def batch(first_chunk, nb):
            ids, mds = [], []
            for b in range(nb):
                off = base + first_chunk * ch + b * ch
                ids.append(pltpu.async_copy(
                    col_hbm.at[pl.ds(off, ch)], idxs[b], sem_i))
                mds.append(pltpu.async_copy(
                    m_hbm.at[pl.ds(off, ch)], bufs[b], sem_m))
            for b in range(nb):
                ids[b].wait()
                mds[b].wait()
                pltpu.async_copy(bufs[b], acc_sh.at[idxs[b]], sem_s,
                                 add=True)

        def drain(nb):
            for b in range(nb):
                pltpu.make_async_copy(
                    bufs[b], acc_sh.at[idxs[b]], sem_s).wait()

        def outer(g, _):
            @pl.when(g > 0)
            def _():
                drain(k)
            batch(g * k, k)
            return 0

        lax.fori_loop(0, n_outer, outer, 0)
        drain(k)
        if rem:
            batch(n_outer * k, rem)
            drain(rem)
        plsc.subcore_barrier()
        pltpu.sync_copy(acc_sh.at[pl.ds(sid * RPS, RPS)],
                        out_hbm.at[cid, pl.ds(sid * RPS, RPS)])

    return scatter


E_A = 128000
E_B = 96000
E_C = 96000
_sc_gather_a = _make_gather(E_A, 80, 10)
_sc_gather_b = _make_gather(E_B, 120, 5)
_sc_gather_c = _make_gather(E_C, 120, 5)
_sc_scatter_a = _make_scatter(E_A, 80, 4)
_sc_scatter_b = _make_scatter(E_B, 40, 8)
_sc_scatter_c = _make_scatter(E_C, 40, 8)


def _dot(a, b):
    return jnp.dot(a, b, preferred_element_type=jnp.float32)


def _edge_body(scale_ref, edge_ref, fsum_ref, We_ref, eW1_ref, eW2_ref,
               n1W1_ref, n1W2_ref, be_ref, eb1_ref, eb2_ref, n1b1_ref,
               n1b2_ref, newe_ref, m_ref):
    e = edge_ref[...].astype(jnp.float32)
    h = jnp.maximum(_dot(e, We_ref[...]) + be_ref[...] + fsum_ref[...], 0.0)
    t = jnp.maximum(_dot(h, eW1_ref[...]) + eb1_ref[...], 0.0)
    ne = scale_ref[0] * e + _dot(t, eW2_ref[...]) + eb2_ref[...]
    u = jnp.maximum(_dot(ne, n1W1_ref[...]) + n1b1_ref[...], 0.0)
    newe_ref[...] = ne.astype(newe_ref.dtype)
    m_ref[...] = _dot(u, n1W2_ref[...]) + n1b2_ref[...]


_W_SPEC = pl.BlockSpec((H, H), lambda i: (0, 0))
_B_SPEC = pl.BlockSpec((1, H), lambda i: (0, 0))
_S_SPEC = pl.BlockSpec(memory_space=pltpu.SMEM)

_WEIGHT_SPECS = [_W_SPEC, _W_SPEC, _W_SPEC, _W_SPEC, _W_SPEC,
                 _B_SPEC, _B_SPEC, _B_SPEC, _B_SPEC, _B_SPEC]


def _edge_first(n_sub, blk_off):
    nblk = n_sub // EBLK
    return pl.pallas_call(
        _edge_body,
        grid=(nblk,),
        in_specs=[_S_SPEC,
                  pl.BlockSpec((EBLK, H), lambda i: (i + blk_off, 0)),
                  pl.BlockSpec((EBLK, H), lambda i: (i, 0))] + _WEIGHT_SPECS,
        out_specs=[pl.BlockSpec((EBLK, H), lambda i: (i, 0)),
                   pl.BlockSpec((EBLK, H), lambda i: (i, 0))],
        out_shape=[jax.ShapeDtypeStruct((n_sub, H), jnp.bfloat16),
                   jax.ShapeDtypeStruct((n_sub, H), jnp.float32)],
    )


def _edge_mid(n_sub):
    nblk = n_sub // EBLK
    return pl.pallas_call(
        _edge_body,
        grid=(nblk,),
        in_specs=[_S_SPEC,
                  pl.BlockSpec((EBLK, H), lambda i: (i, 0)),
                  pl.BlockSpec((EBLK, H), lambda i: (i, 0))] + _WEIGHT_SPECS,
        out_specs=[pl.BlockSpec((EBLK, H), lambda i: (i, 0)),
                   pl.BlockSpec((EBLK, H), lambda i: (i, 0))],
        out_shape=[jax.ShapeDtypeStruct((n_sub, H), jnp.bfloat16),
                   jax.ShapeDtypeStruct((n_sub, H), jnp.float32)],
    )


def _edge_last_first_part(n_sub, blk_off):
    nblk = n_sub // EBLK

    def body(scale_ref, edge_ref, fsum_ref, *rest):
        _edge_body(scale_ref, edge_ref, fsum_ref, *rest)

    return pl.pallas_call(
        body,
        grid=(nblk,),
        in_specs=[_S_SPEC,
                  pl.BlockSpec((EBLK, H), lambda i: (i, 0)),
                  pl.BlockSpec((EBLK, H), lambda i: (i, 0))] + _WEIGHT_SPECS,
        out_specs=[pl.BlockSpec((EBLK, H), lambda i: (i + blk_off, 0)),
                   pl.BlockSpec((EBLK, H), lambda i: (i, 0))],
        out_shape=[jax.ShapeDtypeStruct((N_EDGES, H), jnp.float32),
                   jax.ShapeDtypeStruct((n_sub, H), jnp.float32)],
    )


def _edge_last_next_part(n_sub, blk_off):
    nblk = n_sub // EBLK

    def body(scale_ref, edge_ref, fsum_ref, We_ref, eW1_ref, eW2_ref,
             n1W1_ref, n1W2_ref, be_ref, eb1_ref, eb2_ref, n1b1_ref,
             n1b2_ref, acc_ref, newe_ref, m_ref):
        del acc_ref
        _edge_body(scale_ref, edge_ref, fsum_ref, We_ref, eW1_ref, eW2_ref,
                   n1W1_ref, n1W2_ref, be_ref, eb1_ref, eb2_ref, n1b1_ref,
                   n1b2_ref, newe_ref, m_ref)

    return pl.pallas_call(
        body,
        grid=(nblk,),
        in_specs=[_S_SPEC,
                  pl.BlockSpec((EBLK, H), lambda i: (i, 0)),
                  pl.BlockSpec((EBLK, H), lambda i: (i, 0))] + _WEIGHT_SPECS
        + [pl.BlockSpec(memory_space=pltpu.HBM)],
        out_specs=[pl.BlockSpec((EBLK, H), lambda i: (i + blk_off, 0)),
                   pl.BlockSpec((EBLK, H), lambda i: (i, 0))],
        out_shape=[jax.ShapeDtypeStruct((N_EDGES, H), jnp.float32),
                   jax.ShapeDtypeStruct((n_sub, H), jnp.float32)],
        input_output_aliases={13: 0},
    )


_E_SIZES = (E_A, E_B, E_C)
_E_OFFS = (0, E_A // EBLK, (E_A + E_B) // EBLK)
_edge_first_calls = [_edge_first(s, o) for s, o in zip(_E_SIZES, _E_OFFS)]
_edge_mid_calls = [_edge_mid(s) for s in _E_SIZES]
_edge_last_calls = [
    _edge_last_first_part(_E_SIZES[0], _E_OFFS[0]),
    _edge_last_next_part(_E_SIZES[1], _E_OFFS[1]),
    _edge_last_next_part(_E_SIZES[2], _E_OFFS[2]),
]


def _node_body_proj(scale_ref, agga_ref, aggb_ref, aggc_ref, x_ref, n2W1_ref,
                    n2W2_ref, Wni_ref, Wno_ref, n2b1_ref, n2b2_ref, xnew_ref,
                    xni_ref, xno_ref):
    agg = (agga_ref[0] + agga_ref[1] + aggb_ref[0] + aggb_ref[1]
           + aggc_ref[0] + aggc_ref[1])
    t = jnp.maximum(_dot(agg, n2W1_ref[...]) + n2b1_ref[...], 0.0)
    xn = scale_ref[0] * x_ref[...] + _dot(t, n2W2_ref[...]) + n2b2_ref[...]
    xnew_ref[...] = xn
    xni_ref[...] = _dot(xn, Wni_ref[...])
    xno_ref[...] = _dot(xn, Wno_ref[...])


def _node_body_last(scale_ref, agga_ref, aggb_ref, aggc_ref, x_ref, n2W1_ref,
                    n2W2_ref, n2b1_ref, n2b2_ref, xnew_ref):
    agg = (agga_ref[0] + agga_ref[1] + aggb_ref[0] + aggb_ref[1]
           + aggc_ref[0] + aggc_ref[1])
    t = jnp.maximum(_dot(agg, n2W1_ref[...]) + n2b1_ref[...], 0.0)
    xnew_ref[...] = scale_ref[0] * x_ref[...] + _dot(t, n2W2_ref[...]) \
        + n2b2_ref[...]


_AGG_SPEC = pl.BlockSpec((NC, NBLK, H), lambda i: (0, i, 0))
_N_SPEC = pl.BlockSpec((NBLK, H), lambda i: (i, 0))
_NODE_SHAPE = jax.ShapeDtypeStruct((N_NODES, H), jnp.float32)

_node_call_proj = pl.pallas_call(
    _node_body_proj,
    grid=(N_NODES // NBLK,),
    in_specs=[_S_SPEC, _AGG_SPEC, _AGG_SPEC, _AGG_SPEC, _N_SPEC,
              _W_SPEC, _W_SPEC, _W_SPEC, _W_SPEC, _B_SPEC, _B_SPEC],
    out_specs=[_N_SPEC, _N_SPEC, _N_SPEC],
    out_shape=[_NODE_SHAPE, _NODE_SHAPE, _NODE_SHAPE],
)

_node_call_last = pl.pallas_call(
    _node_body_last,
    grid=(N_NODES // NBLK,),
    in_specs=[_S_SPEC, _AGG_SPEC, _AGG_SPEC, _AGG_SPEC, _N_SPEC, _W_SPEC,
              _W_SPEC, _B_SPEC, _B_SPEC],
    out_specs=_N_SPEC,
    out_shape=_NODE_SHAPE,
)


def _proj_body(x_ref, Wni_ref, Wno_ref, xni_ref, xno_ref):
    x = x_ref[...]
    xni_ref[...] = _dot(x, Wni_ref[...])
    xno_ref[...] = _dot(x, Wno_ref[...])


_proj_call = pl.pallas_call(
    _proj_body,
    grid=(N_NODES // NBLK,),
    in_specs=[_N_SPEC, _W_SPEC, _W_SPEC],
    out_specs=[_N_SPEC, _N_SPEC],
    out_shape=[_NODE_SHAPE, _NODE_SHAPE],
)


def kernel(x, edge_index, edge_attr, We, Wni, Wno, eW1, eW2, n1W1, n1W2,
           n2W1, n2W2, be, eb1, eb2, n1b1, n1b2, n2b1, n2b2, eps_e, eps_n):
    row_a, col_a = edge_index[0, :E_A], edge_index[1, :E_A]
    row_b, col_b = edge_index[0, E_A:E_A + E_B], edge_index[1, E_A:E_A + E_B]
    row_c, col_c = edge_index[0, E_A + E_B:], edge_index[1, E_A + E_B:]
    zeros_nh = jnp.zeros((N_PAD, H), jnp.float32)

    xni, xno = _proj_call(x, Wni[0], Wno[0])
    ea = eb = ec = None
    edge_out = None
    for i in range(DEPTH):
        scale_e = (1.0 + eps_e[i]).reshape((1,))
        wargs = (We[i], eW1[i], eW2[i], n1W1[i], n1W2[i],
                 be[i].reshape(1, H), eb1[i].reshape(1, H),
                 eb2[i].reshape(1, H), n1b1[i].reshape(1, H),
                 n1b2[i].reshape(1, H))

        def ecall(part, part_state, fs):
            if i == 0:
                return _edge_first_calls[part](scale_e, edge_attr, fs,
                                               *wargs)
            if i + 1 < DEPTH:
                return _edge_mid_calls[part](scale_e, part_state, fs, *wargs)
            if part == 0:
                return _edge_last_calls[0](scale_e, part_state, fs, *wargs)
            return _edge_last_calls[part](scale_e, part_state, fs, *wargs,
                                          edge_out)

        fs_a = _sc_gather_a(xni, xno, row_a, col_a)
        fs_b = _sc_gather_b(xni, xno, row_b, col_b)
        na, m_a = ecall(0, ea, fs_a)
        if i + 1 == DEPTH:
            edge_out = na
        else:
            ea = na
        fs_c = _sc_gather_c(xni, xno, row_c, col_c)
        nb, m_b = ecall(1, eb, fs_b)
        if i + 1 == DEPTH:
            edge_out = nb
        else:
            eb = nb
        agg_a = _sc_scatter_a(m_a, col_a, zeros_nh)
        nc, m_c = ecall(2, ec, fs_c)
        if i + 1 == DEPTH:
            edge_out = nc
        else:
            ec = nc
        agg_b = _sc_scatter_b(m_b, col_b, zeros_nh)
        agg_c = _sc_scatter_c(m_c, col_c, zeros_nh)
        scale_n = (1.0 + eps_n[i]).reshape((1,))
        if i + 1 < DEPTH:
            x, xni, xno = _node_call_proj(
                scale_n, agg_a, agg_b, agg_c, x, n2W1[i], n2W2[i],
                Wni[i + 1], Wno[i + 1], n2b1[i].reshape(1, H),
                n2b2[i].reshape(1, H))
        else:
            x = _node_call_last(
                scale_n, agg_a, agg_b, agg_c, x, n2W1[i], n2W2[i],
                n2b1[i].reshape(1, H), n2b2[i].reshape(1, H))
    return (x, edge_out)


# speedup vs baseline: 1.3482x; 1.0078x over previous
import functools

import jax
import jax.numpy as jnp
from jax import lax
from jax.experimental import pallas as pl
from jax.experimental.pallas import tpu as pltpu
from jax.experimental.pallas import tpu_sc as plsc

N_NODES = 10000
N_EDGES = 320000
H = 128
DEPTH = 3

NC = 2
NS = 16
NW = NC * NS
EPW = N_EDGES // NW
CH = 80
NCHUNK = EPW // CH
N_PAD = 10112
RPS = N_PAD // NS

EBLK = 2000
NBLK = 2000

_mesh = plsc.VectorSubcoreMesh(core_axis_name="c", subcore_axis_name="s")


K = 5


def _make_gather(n_edges, ch, k):
    epw = n_edges // NW
    n_chunks = epw // ch
    n_outer = n_chunks // k
    rem = n_chunks - n_outer * k

    @functools.partial(
        pl.kernel,
        out_type=jax.ShapeDtypeStruct((n_edges, H), jnp.float32),
        mesh=_mesh,
        scratch_types=[
            pltpu.VMEM((epw,), jnp.int32),
            pltpu.VMEM((epw,), jnp.int32),
            pltpu.VMEM((k, ch, H), jnp.float32),
            pltpu.SemaphoreType.DMA,
            pltpu.SemaphoreType.DMA,
            pltpu.SemaphoreType.DMA,
        ],
    )
    def gather(xni_hbm, xno_hbm, row_hbm, col_hbm, out_hbm,
               ridx_v, cidx_v, bufs, sem_g, sem_a, sem_w):
        wid = lax.axis_index("s") * NC + lax.axis_index("c")
        base = wid * epw
        pltpu.sync_copy(row_hbm.at[pl.ds(base, epw)], ridx_v)
        pltpu.sync_copy(col_hbm.at[pl.ds(base, epw)], cidx_v)

        def drain_wb(first_chunk, nb):
            for b in range(nb):
                off = base + (first_chunk + b) * ch
                pltpu.make_async_copy(
                    bufs.at[b], out_hbm.at[pl.ds(off, ch)], sem_w).wait()

        def batch(first_chunk, nb):
            gds, ads = [], []
            for b in range(nb):
                c = (first_chunk + b) * ch
                gds.append(pltpu.async_copy(
                    xni_hbm.at[ridx_v.at[pl.ds(c, ch)]], bufs.at[b], sem_g))
            for b in range(nb):
                c = (first_chunk + b) * ch
                gds[b].wait()
                ads.append(pltpu.async_copy(
                    xno_hbm.at[cidx_v.at[pl.ds(c, ch)]], bufs.at[b], sem_a,
                    add=True))
            for b in range(nb):
                off = base + (first_chunk + b) * ch
                ads[b].wait()
                pltpu.async_copy(bufs.at[b], out_hbm.at[pl.ds(off, ch)],
                                 sem_w)

        def outer(g, _):
            @pl.when(g > 0)
            def _():
                drain_wb((g - 1) * k, k)
            batch(g * k, k)
            return 0

        lax.fori_loop(0, n_outer, outer, 0)
        drain_wb((n_outer - 1) * k, k)
        if rem:
            batch(n_outer * k, rem)
            drain_wb(n_outer * k, rem)

    return gather


def _make_scatter(n_edges, ch, k):
    epw = n_edges // NW
    n_chunks = epw // ch
    n_outer = n_chunks // k
    rem = n_chunks - n_outer * k

    @functools.partial(
        pl.kernel,
        out_type=jax.ShapeDtypeStruct((NC, N_PAD, H), jnp.float32),
        mesh=_mesh,
        scratch_types=(
            [pltpu.VMEM((ch,), jnp.int32)] * k
            + [pltpu.VMEM((ch, H), jnp.float32)] * k
            + [
                pltpu.VMEM_SHARED((N_PAD, H), jnp.float32),
                pltpu.SemaphoreType.DMA,
                pltpu.SemaphoreType.DMA,
                pltpu.SemaphoreType.DMA,
            ]
        ),
    )
    def scatter(m_hbm, col_hbm, zero_hbm, out_hbm, *rest):
        idxs = rest[:k]
        bufs = rest[k:2 * k]
        acc_sh, sem_i, sem_m, sem_s = rest[2 * k:]
        cid = lax.axis_index("c")
        sid = lax.axis_index("s")
        wid = sid * NC + cid
        pltpu.sync_copy(zero_hbm.at[pl.ds(sid * RPS, RPS)],
                        acc_sh.at[pl.ds(sid * RPS, RPS)])
        plsc.subcore_barrier()
        base = wid * epw

        def batch(first_chunk, nb):
            ids, mds = [], []
            for b in range(nb):
                off = base + first_chunk * ch + b * ch
                ids.append(pltpu.async_copy(
                    col_hbm.at[pl.ds(off, ch)], idxs[b], sem_i))
                mds.append(pltpu.async_copy(
                    m_hbm.at[pl.ds(off, ch)], bufs[b], sem_m))
            for b in range(nb):
                ids[b].wait()
                mds[b].wait()
                pltpu.async_copy(bufs[b], acc_sh.at[idxs[b]], sem_s,
                                 add=True)

        def drain(nb):
            for b in range(nb):
                pltpu.make_async_copy(
                    bufs[b], acc_sh.at[idxs[b]], sem_s).wait()

        def outer(g, _):
            @pl.when(g > 0)
            def _():
                drain(k)
            batch(g * k, k)
            return 0

        lax.fori_loop(0, n_outer, outer, 0)
        drain(k)
        if rem:
            batch(n_outer * k, rem)
            drain(rem)
        plsc.subcore_barrier()
        pltpu.sync_copy(acc_sh.at[pl.ds(sid * RPS, RPS)],
                        out_hbm.at[cid, pl.ds(sid * RPS, RPS)])

    return scatter


E_A = 96000
E_B = 128000
E_C = 96000
_sc_gather_a = _make_gather(E_A, 120, 5)
_sc_gather_b = _make_gather(E_B, 80, 10)
_sc_gather_c = _make_gather(E_C, 120, 5)
_sc_scatter_a = _make_scatter(E_A, 40, 8)
_sc_scatter_b = _make_scatter(E_B, 80, 4)
_sc_scatter_c = _make_scatter(E_C, 40, 8)


def _dot(a, b):
    return jnp.dot(a, b, preferred_element_type=jnp.float32)


def _edge_body(scale_ref, edge_ref, fsum_ref, We_ref, eW1_ref, eW2_ref,
               n1W1_ref, n1W2_ref, be_ref, eb1_ref, eb2_ref, n1b1_ref,
               n1b2_ref, newe_ref, m_ref):
    e = edge_ref[...].astype(jnp.float32)
    h = jnp.maximum(_dot(e, We_ref[...]) + be_ref[...] + fsum_ref[...], 0.0)
    t = jnp.maximum(_dot(h, eW1_ref[...]) + eb1_ref[...], 0.0)
    ne = scale_ref[0] * e + _dot(t, eW2_ref[...]) + eb2_ref[...]
    u = jnp.maximum(_dot(ne, n1W1_ref[...]) + n1b1_ref[...], 0.0)
    newe_ref[...] = ne.astype(newe_ref.dtype)
    m_ref[...] = _dot(u, n1W2_ref[...]) + n1b2_ref[...]


_W_SPEC = pl.BlockSpec((H, H), lambda i: (0, 0))
_B_SPEC = pl.BlockSpec((1, H), lambda i: (0, 0))
_S_SPEC = pl.BlockSpec(memory_space=pltpu.SMEM)

_WEIGHT_SPECS = [_W_SPEC, _W_SPEC, _W_SPEC, _W_SPEC, _W_SPEC,
                 _B_SPEC, _B_SPEC, _B_SPEC, _B_SPEC, _B_SPEC]


def _edge_first(n_sub, blk_off):
    nblk = n_sub // EBLK
    return pl.pallas_call(
        _edge_body,
        grid=(nblk,),
        in_specs=[_S_SPEC,
                  pl.BlockSpec((EBLK, H), lambda i: (i + blk_off, 0)),
                  pl.BlockSpec((EBLK, H), lambda i: (i, 0))] + _WEIGHT_SPECS,
        out_specs=[pl.BlockSpec((EBLK, H), lambda i: (i, 0)),
                   pl.BlockSpec((EBLK, H), lambda i: (i, 0))],
        out_shape=[jax.ShapeDtypeStruct((n_sub, H), jnp.bfloat16),
                   jax.ShapeDtypeStruct((n_sub, H), jnp.float32)],
    )


def _edge_mid(n_sub):
    nblk = n_sub // EBLK
    return pl.pallas_call(
        _edge_body,
        grid=(nblk,),
        in_specs=[_S_SPEC,
                  pl.BlockSpec((EBLK, H), lambda i: (i, 0)),
                  pl.BlockSpec((EBLK, H), lambda i: (i, 0))] + _WEIGHT_SPECS,
        out_specs=[pl.BlockSpec((EBLK, H), lambda i: (i, 0)),
                   pl.BlockSpec((EBLK, H), lambda i: (i, 0))],
        out_shape=[jax.ShapeDtypeStruct((n_sub, H), jnp.bfloat16),
                   jax.ShapeDtypeStruct((n_sub, H), jnp.float32)],
    )


def _edge_last_first_part(n_sub, blk_off):
    nblk = n_sub // EBLK

    def body(scale_ref, edge_ref, fsum_ref, *rest):
        _edge_body(scale_ref, edge_ref, fsum_ref, *rest)

    return pl.pallas_call(
        body,
        grid=(nblk,),
        in_specs=[_S_SPEC,
                  pl.BlockSpec((EBLK, H), lambda i: (i, 0)),
                  pl.BlockSpec((EBLK, H), lambda i: (i, 0))] + _WEIGHT_SPECS,
        out_specs=[pl.BlockSpec((EBLK, H), lambda i: (i + blk_off, 0)),
                   pl.BlockSpec((EBLK, H), lambda i: (i, 0))],
        out_shape=[jax.ShapeDtypeStruct((N_EDGES, H), jnp.float32),
                   jax.ShapeDtypeStruct((n_sub, H), jnp.float32)],
    )


def _edge_last_next_part(n_sub, blk_off):
    nblk = n_sub // EBLK

    def body(scale_ref, edge_ref, fsum_ref, We_ref, eW1_ref, eW2_ref,
             n1W1_ref, n1W2_ref, be_ref, eb1_ref, eb2_ref, n1b1_ref,
             n1b2_ref, acc_ref, newe_ref, m_ref):
        del acc_ref
        _edge_body(scale_ref, edge_ref, fsum_ref, We_ref, eW1_ref, eW2_ref,
                   n1W1_ref, n1W2_ref, be_ref, eb1_ref, eb2_ref, n1b1_ref,
                   n1b2_ref, newe_ref, m_ref)

    return pl.pallas_call(
        body,
        grid=(nblk,),
        in_specs=[_S_SPEC,
                  pl.BlockSpec((EBLK, H), lambda i: (i, 0)),
                  pl.BlockSpec((EBLK, H), lambda i: (i, 0))] + _WEIGHT_SPECS
        + [pl.BlockSpec(memory_space=pltpu.HBM)],
        out_specs=[pl.BlockSpec((EBLK, H), lambda i: (i + blk_off, 0)),
                   pl.BlockSpec((EBLK, H), lambda i: (i, 0))],
        out_shape=[jax.ShapeDtypeStruct((N_EDGES, H), jnp.float32),
                   jax.ShapeDtypeStruct((n_sub, H), jnp.float32)],
        input_output_aliases={13: 0},
    )


_E_SIZES = (E_A, E_B, E_C)
_E_OFFS = (0, E_A // EBLK, (E_A + E_B) // EBLK)
_edge_first_calls = [_edge_first(s, o) for s, o in zip(_E_SIZES, _E_OFFS)]
_edge_mid_calls = [_edge_mid(s) for s in _E_SIZES]
_edge_last_calls = [
    _edge_last_first_part(_E_SIZES[0], _E_OFFS[0]),
    _edge_last_next_part(_E_SIZES[1], _E_OFFS[1]),
    _edge_last_next_part(_E_SIZES[2], _E_OFFS[2]),
]


def _node_body_proj(scale_ref, agga_ref, aggb_ref, aggc_ref, x_ref, n2W1_ref,
                    n2W2_ref, Wni_ref, Wno_ref, n2b1_ref, n2b2_ref, xnew_ref,
                    xni_ref, xno_ref):
    agg = (agga_ref[0] + agga_ref[1] + aggb_ref[0] + aggb_ref[1]
           + aggc_ref[0] + aggc_ref[1])
    t = jnp.maximum(_dot(agg, n2W1_ref[...]) + n2b1_ref[...], 0.0)
    xn = scale_ref[0] * x_ref[...] + _dot(t, n2W2_ref[...]) + n2b2_ref[...]
    xnew_ref[...] = xn
    xni_ref[...] = _dot(xn, Wni_ref[...])
    xno_ref[...] = _dot(xn, Wno_ref[...])


def _node_body_last(scale_ref, agga_ref, aggb_ref, aggc_ref, x_ref, n2W1_ref,
                    n2W2_ref, n2b1_ref, n2b2_ref, xnew_ref):
    agg = (agga_ref[0] + agga_ref[1] + aggb_ref[0] + aggb_ref[1]
           + aggc_ref[0] + aggc_ref[1])
    t = jnp.maximum(_dot(agg, n2W1_ref[...]) + n2b1_ref[...], 0.0)
    xnew_ref[...] = scale_ref[0] * x_ref[...] + _dot(t, n2W2_ref[...]) \
        + n2b2_ref[...]


_AGG_SPEC = pl.BlockSpec((NC, NBLK, H), lambda i: (0, i, 0))
_N_SPEC = pl.BlockSpec((NBLK, H), lambda i: (i, 0))
_NODE_SHAPE = jax.ShapeDtypeStruct((N_NODES, H), jnp.float32)

_node_call_proj = pl.pallas_call(
    _node_body_proj,
    grid=(N_NODES // NBLK,),
    in_specs=[_S_SPEC, _AGG_SPEC, _AGG_SPEC, _AGG_SPEC, _N_SPEC,
              _W_SPEC, _W_SPEC, _W_SPEC, _W_SPEC, _B_SPEC, _B_SPEC],
    out_specs=[_N_SPEC, _N_SPEC, _N_SPEC],
    out_shape=[_NODE_SHAPE, _NODE_SHAPE, _NODE_SHAPE],
)

_node_call_last = pl.pallas_call(
    _node_body_last,
    grid=(N_NODES // NBLK,),
    in_specs=[_S_SPEC, _AGG_SPEC, _AGG_SPEC, _AGG_SPEC, _N_SPEC, _W_SPEC,
              _W_SPEC, _B_SPEC, _B_SPEC],
    out_specs=_N_SPEC,
    out_shape=_NODE_SHAPE,
)


def _proj_body(x_ref, Wni_ref, Wno_ref, xni_ref, xno_ref):
    x = x_ref[...]
    xni_ref[...] = _dot(x, Wni_ref[...])
    xno_ref[...] = _dot(x, Wno_ref[...])


_proj_call = pl.pallas_call(
    _proj_body,
    grid=(N_NODES // NBLK,),
    in_specs=[_N_SPEC, _W_SPEC, _W_SPEC],
    out_specs=[_N_SPEC, _N_SPEC],
    out_shape=[_NODE_SHAPE, _NODE_SHAPE],
)


def kernel(x, edge_index, edge_attr, We, Wni, Wno, eW1, eW2, n1W1, n1W2,
           n2W1, n2W2, be, eb1, eb2, n1b1, n1b2, n2b1, n2b2, eps_e, eps_n):
    row_a, col_a = edge_index[0, :E_A], edge_index[1, :E_A]
    row_b, col_b = edge_index[0, E_A:E_A + E_B], edge_index[1, E_A:E_A + E_B]
    row_c, col_c = edge_index[0, E_A + E_B:], edge_index[1, E_A + E_B:]
    zeros_nh = jnp.zeros((N_PAD, H), jnp.float32)

    xni, xno = _proj_call(x, Wni[0], Wno[0])
    ea = eb = ec = None
    edge_out = None
    for i in range(DEPTH):
        scale_e = (1.0 + eps_e[i]).reshape((1,))
        wargs = (We[i], eW1[i], eW2[i], n1W1[i], n1W2[i],
                 be[i].reshape(1, H), eb1[i].reshape(1, H),
                 eb2[i].reshape(1, H), n1b1[i].reshape(1, H),
                 n1b2[i].reshape(1, H))

        def ecall(part, part_state, fs):
            if i == 0:
                return _edge_first_calls[part](scale_e, edge_attr, fs,
                                               *wargs)
            if i + 1 < DEPTH:
                return _edge_mid_calls[part](scale_e, part_state, fs, *wargs)
            if part == 0:
                return _edge_last_calls[0](scale_e, part_state, fs, *wargs)
            return _edge_last_calls[part](scale_e, part_state, fs, *wargs,
                                          edge_out)

        fs_a = _sc_gather_a(xni, xno, row_a, col_a)
        fs_b = _sc_gather_b(xni, xno, row_b, col_b)
        na, m_a = ecall(0, ea, fs_a)
        if i + 1 == DEPTH:
            edge_out = na
        else:
            ea = na
        fs_c = _sc_gather_c(xni, xno, row_c, col_c)
        nb, m_b = ecall(1, eb, fs_b)
        if i + 1 == DEPTH:
            edge_out = nb
        else:
            eb = nb
        agg_a = _sc_scatter_a(m_a, col_a, zeros_nh)
        nc, m_c = ecall(2, ec, fs_c)
        if i + 1 == DEPTH:
            edge_out = nc
        else:
            ec = nc
        agg_b = _sc_scatter_b(m_b, col_b, zeros_nh)
        agg_c = _sc_scatter_c(m_c, col_c, zeros_nh)
        scale_n = (1.0 + eps_n[i]).reshape((1,))
        if i + 1 < DEPTH:
            x, xni, xno = _node_call_proj(
                scale_n, agg_a, agg_b, agg_c, x, n2W1[i], n2W2[i],
                Wni[i + 1], Wno[i + 1], n2b1[i].reshape(1, H),
                n2b2[i].reshape(1, H))
        else:
            x = _node_call_last(
                scale_n, agg_a, agg_b, agg_c, x, n2W1[i], n2W2[i],
                n2b1[i].reshape(1, H), n2b2[i].reshape(1, H))
    return (x, edge_out)
